# Initial kernel scaffold; baseline (speedup 1.0000x reference)
import jax
import jax.numpy as jnp
from jax.experimental import pallas as pl


def kernel(x, edge_index, edge_attr, batch, Wn1, bn1, Wn2, bn2, We1, be1, We2, be2, Wfc, bfc):
    raise NotImplementedError("write your pallas kernel here")


# speedup vs baseline: 1.4664x; 1.4664x over previous
import functools

import jax
import jax.numpy as jnp
from jax.experimental import pallas as pl


def _mm_scale_body(x_ref, w_ref, scale_ref, out_ref):
    out_ref[...] = scale_ref[...] * jnp.dot(
        x_ref[...], w_ref[...], preferred_element_type=jnp.float32)


def _mm_scale(x, w, scale, block_rows):
    n, k = x.shape
    m = w.shape[1]
    grid = n // block_rows
    return pl.pallas_call(
        _mm_scale_body,
        grid=(grid,),
        in_specs=[
            pl.BlockSpec((block_rows, k), lambda i: (i, 0)),
            pl.BlockSpec((k, m), lambda i: (0, 0)),
            pl.BlockSpec((block_rows, 1), lambda i: (i, 0)),
        ],
        out_specs=pl.BlockSpec((block_rows, m), lambda i: (i, 0)),
        out_shape=jax.ShapeDtypeStruct((n, m), jnp.float32),
    )(x, w, scale)


def _node_reduce_body(s_ref, h1s_ref, dis_ref, wn_ref, b_ref, out_ref):
    @pl.when(pl.program_id(0) == 0)
    def _():
        out_ref[...] = jnp.zeros_like(out_ref)

    h = jax.nn.sigmoid(dis_ref[...] * (s_ref[...] + h1s_ref[...]) + b_ref[...])
    out_ref[...] += jnp.dot(wn_ref[...].T, h, preferred_element_type=jnp.float32)


def _node_reduce(s, h1s, dis, wn, b, block_rows):
    n, d = s.shape
    grid = n // block_rows
    return pl.pallas_call(
        _node_reduce_body,
        grid=(grid,),
        in_specs=[
            pl.BlockSpec((block_rows, d), lambda i: (i, 0)),
            pl.BlockSpec((block_rows, d), lambda i: (i, 0)),
            pl.BlockSpec((block_rows, 1), lambda i: (i, 0)),
            pl.BlockSpec((block_rows, 1), lambda i: (i, 0)),
            pl.BlockSpec((1, d), lambda i: (0, 0)),
        ],
        out_specs=pl.BlockSpec((1, d), lambda i: (0, 0)),
        out_shape=jax.ShapeDtypeStruct((1, d), jnp.float32),
    )(s, h1s, dis, wn, b)


def _edge_reduce_body(z_ref, lam_ref, b_ref, out_ref):
    @pl.when(pl.program_id(0) == 0)
    def _():
        out_ref[...] = jnp.zeros_like(out_ref)

    e = jax.nn.sigmoid(z_ref[...] + b_ref[...])
    out_ref[...] += jnp.dot(lam_ref[...].T, e, preferred_element_type=jnp.float32)


def _edge_reduce(z, lam, b, block_rows):
    n, d = z.shape
    grid = n // block_rows
    return pl.pallas_call(
        _edge_reduce_body,
        grid=(grid,),
        in_specs=[pl.BlockSpec((block_rows, d), lambda i: (i, 0)),
                  pl.BlockSpec((block_rows, 1), lambda i: (i, 0)),
                  pl.BlockSpec((1, d), lambda i: (0, 0))],
        out_specs=pl.BlockSpec((1, d), lambda i: (0, 0)),
        out_shape=jax.ShapeDtypeStruct((1, d), jnp.float32),
    )(z, lam, b)


def _tail_body(ns_ref, es_ref, wn2_ref, we2_ref, wfc_ref, bn2_ref, be2_ref,
               bfc_ref, nn_ref, ee_ref, out_ref):
    node_rep = jnp.dot(ns_ref[...], wn2_ref[...],
                       preferred_element_type=jnp.float32) / nn_ref[0, 0] + bn2_ref[...]
    edge_rep = jnp.dot(es_ref[...], we2_ref[...],
                       preferred_element_type=jnp.float32) / ee_ref[0, 0] + be2_ref[...]
    rep = jnp.concatenate([node_rep, edge_rep], axis=-1)
    out_ref[...] = jnp.dot(rep, wfc_ref[...],
                           preferred_element_type=jnp.float32) + bfc_ref[...]


def _tail(nodesum, edgesum, Wn2, We2, Wfc, bn2, be2, bfc, n_nodes, n_edges):
    nn = jnp.full((1, 1), float(n_nodes), jnp.float32)
    ee = jnp.full((1, 1), float(n_edges), jnp.float32)
    full = lambda shape: pl.BlockSpec(shape, lambda: (0,) * len(shape))
    return pl.pallas_call(
        _tail_body,
        in_specs=[full(nodesum.shape), full(edgesum.shape), full(Wn2.shape),
                  full(We2.shape), full(Wfc.shape), full((1, Wn2.shape[1])),
                  full((1, We2.shape[1])), full((1, Wfc.shape[1])),
                  full((1, 1)), full((1, 1))],
        out_specs=full((1, Wfc.shape[1])),
        out_shape=jax.ShapeDtypeStruct((1, Wfc.shape[1]), jnp.float32),
    )(nodesum, edgesum, Wn2, We2, Wfc, bn2[None, :], be2[None, :], bfc[None, :],
      nn, ee)


def kernel(x, edge_index, edge_attr, batch, Wn1, bn1, Wn2, bn2, We1, be1,
           We2, be2, Wfc, bfc):
    N = x.shape[0]
    E = edge_index.shape[1]
    row, col = edge_index[0], edge_index[1]

    deg = jnp.zeros((N,), jnp.float32).at[col].add(1.0) + 1.0
    dis = deg ** -0.5
    h1s = _mm_scale(x, Wn1, dis[:, None], 1000)
    s = jnp.zeros((N, Wn1.shape[1]), jnp.float32).at[col].add(h1s[row])
    q = jnp.zeros((N,), jnp.float32).at[row].add(dis[col])
    wnode = dis * q + dis * dis
    nodesum = _node_reduce(s, h1s, dis[:, None], wnode[:, None],
                           bn1[None, :], 1000)

    u0, u1 = row, col
    nodes = edge_index.T.reshape(-1)
    order = jnp.argsort(nodes, stable=True)
    nodes_s = nodes[order]
    idx = jnp.arange(2 * E, dtype=jnp.int32)
    is_start = jnp.concatenate([jnp.array([True]), nodes_s[1:] != nodes_s[:-1]])
    start = jax.lax.cummax(jnp.where(is_start, idx, 0))
    rank = idx - start
    cnt = jnp.zeros((N,), jnp.int32).at[nodes].add(1)
    kcap = jnp.minimum(cnt[nodes_s], 100)
    counted = jnp.zeros((2 * E,), bool).at[order].set(rank < kcap)
    c0, c1 = counted[0::2], counted[1::2]
    selfloop = u0 == u1
    w0 = (c0 | (selfloop & c1)).astype(jnp.float32)
    w1 = (c1 & (~selfloop)).astype(jnp.float32)
    both = c0 & c1
    eligible = both & (~selfloop)
    key = jnp.where(eligible,
                    jnp.minimum(u0, u1) * N + jnp.maximum(u0, u1),
                    N * N)
    p = jnp.argsort(key)
    key_s = key[p]
    gid = jnp.cumsum(jnp.concatenate(
        [jnp.zeros((1,), jnp.int32), (key_s[1:] != key_s[:-1]).astype(jnp.int32)]))
    inv = jnp.argsort(p)
    Fc = jnp.zeros((N,), jnp.float32).at[u0].add(w0).at[u1].add(w1)
    Gc = jax.ops.segment_sum(jnp.ones((E,), jnp.float32), gid,
                             num_segments=E)[gid][inv]
    degA = jnp.where(eligible, Fc[u0] + Fc[u1] - Gc - 1.0,
           jnp.where(both, Fc[u0],
           jnp.where(c0, Fc[u0] - 1.0,
           jnp.where(c1, Fc[u1] - 1.0, 0.0))))
    dise = (degA + 1.0) ** -0.5

    f = _mm_scale(edge_attr, We1, dise[:, None], 8000)
    F = jnp.zeros((N, We1.shape[1]), jnp.float32).at[u0].add(
        f * w0[:, None]).at[u1].add(f * w1[:, None])
    G = jax.ops.segment_sum(f[p], gid, num_segments=E)[gid][inv]

    alpha0 = c0.astype(jnp.float32)
    alpha1 = (eligible | (c1 & ~c0)).astype(jnp.float32)
    alphaf = -((c0 | c1) & ~(both & selfloop)).astype(jnp.float32)
    C = jnp.zeros((N,), jnp.float32).at[u0].add(dise * alpha0).at[u1].add(
        dise * alpha1)
    T = jax.ops.segment_sum(jnp.where(eligible, -dise, 0.0)[p], gid,
                            num_segments=E)[gid][inv]
    lam = dise * (w0 * C[u0] + w1 * C[u1] + T) + dise * dise * (alphaf + 1.0)

    elig_f = eligible.astype(jnp.float32)
    z = (dise * alpha0)[:, None] * F[u0] + (dise * alpha1)[:, None] * F[u1] \
        - (dise * elig_f)[:, None] * G + (dise * (alphaf + 1.0))[:, None] * f
    edgesum = _edge_reduce(z, lam[:, None], be1[None, :], 4000)

    return _tail(nodesum, edgesum, Wn2, We2, Wfc, bn2, be2, bfc, N, E)


# speedup vs baseline: 1.4951x; 1.0196x over previous
import functools

import jax
import jax.numpy as jnp
from jax import lax
from jax.experimental import pallas as pl
from jax.experimental.pallas import tpu as pltpu
from jax.experimental.pallas import tpu_sc as plsc

_NC = 2
_NS = 16
_L = 16


_NPAD = 10240
_HALF = 5120
_REAL_HALF = 5000
_DUMMY = 5100
_EK = 80


def _node_scatter_sc(h1s_p, row, col, d_feat, n_edges):
    epw = n_edges // _NS
    chunks = epw // _EK
    mesh = plsc.VectorSubcoreMesh(core_axis_name="c", subcore_axis_name="s")

    @functools.partial(
        pl.kernel,
        out_type=jax.ShapeDtypeStruct((_NPAD, d_feat), jnp.float32),
        mesh=mesh,
        scratch_types=dict(
            table=pltpu.VMEM_SHARED((_HALF, d_feat), jnp.float32),
            rowflat=pltpu.VMEM((epw,), jnp.int32),
            colflat=pltpu.VMEM((epw,), jnp.int32),
            lidx=pltpu.VMEM((chunks, _EK), jnp.int32),
            rbuf=pltpu.VMEM((_EK, d_feat), jnp.float32),
            zbuf=pltpu.VMEM((_EK, d_feat), jnp.float32),
            sem=pltpu.SemaphoreType.DMA,
        ),
    )
    def k(h1s_hbm, row_hbm, col_hbm, out_hbm, table, rowflat, colflat, lidx,
          rbuf, zbuf, sem):
        cc = lax.axis_index("c")
        tid = lax.axis_index("s")
        base = cc * _REAL_HALF
        nb = d_feat // _L

        def zrow(r, _):
            for j in range(nb):
                zbuf[r, pl.ds(j * _L, _L)] = jnp.zeros((_L,), jnp.float32)
            return 0
        lax.fori_loop(0, _EK, zrow, 0)
        rows_per_tile = _HALF // _NS
        for j in range(rows_per_tile // _EK):
            pltpu.sync_copy(zbuf, table.at[pl.ds(tid * rows_per_tile + j * _EK,
                                                 _EK)])
        plsc.subcore_barrier()

        e0 = tid * epw
        pltpu.sync_copy(row_hbm.at[pl.ds(e0, epw)], rowflat)
        pltpu.sync_copy(col_hbm.at[pl.ds(e0, epw)], colflat)

        def lidx_body(g, _):
            for j in range(_EK // _L):
                cv = colflat[pl.ds(g * _EK + j * _L, _L)]
                lc = cv - base
                m = (lc >= 0) & (lc < _REAL_HALF)
                lidx[g, pl.ds(j * _L, _L)] = jnp.where(m, lc, _DUMMY)
            return 0
        lax.fori_loop(0, chunks, lidx_body, 0)

        def chunk_body(g, _):
            pltpu.async_copy(h1s_hbm.at[rowflat.at[pl.ds(g * _EK, _EK)]],
                             rbuf, sem).wait()
            pltpu.sync_copy(rbuf, table.at[lidx.at[g]], add=True)
            return 0
        lax.fori_loop(0, chunks, chunk_body, 0)
        plsc.subcore_barrier()

        for j in range(rows_per_tile // _EK):
            local = tid * rows_per_tile + j * _EK
            @pl.when(local < _REAL_HALF)
            def _():
                pltpu.sync_copy(table.at[pl.ds(local, _EK)], rbuf)
                pltpu.sync_copy(rbuf, out_hbm.at[pl.ds(base + local, _EK)])

    return k(h1s_p, row, col)


def _mm_scale_body(x_ref, w_ref, scale_ref, out_ref):
    out_ref[...] = scale_ref[...] * jnp.dot(
        x_ref[...], w_ref[...], preferred_element_type=jnp.float32)


def _mm_scale(x, w, scale, block_rows):
    n, k = x.shape
    m = w.shape[1]
    grid = n // block_rows
    return pl.pallas_call(
        _mm_scale_body,
        grid=(grid,),
        in_specs=[
            pl.BlockSpec((block_rows, k), lambda i: (i, 0)),
            pl.BlockSpec((k, m), lambda i: (0, 0)),
            pl.BlockSpec((block_rows, 1), lambda i: (i, 0)),
        ],
        out_specs=pl.BlockSpec((block_rows, m), lambda i: (i, 0)),
        out_shape=jax.ShapeDtypeStruct((n, m), jnp.float32),
    )(x, w, scale)


def _node_reduce_body(s_ref, h1s_ref, dis_ref, wn_ref, b_ref, out_ref):
    @pl.when(pl.program_id(0) == 0)
    def _():
        out_ref[...] = jnp.zeros_like(out_ref)

    h = jax.nn.sigmoid(dis_ref[...] * (s_ref[...] + h1s_ref[...]) + b_ref[...])
    out_ref[...] += jnp.dot(wn_ref[...].T, h, preferred_element_type=jnp.float32)


def _node_reduce(s, h1s, dis, wn, b, block_rows):
    n, d = s.shape
    grid = n // block_rows
    return pl.pallas_call(
        _node_reduce_body,
        grid=(grid,),
        in_specs=[
            pl.BlockSpec((block_rows, d), lambda i: (i, 0)),
            pl.BlockSpec((block_rows, d), lambda i: (i, 0)),
            pl.BlockSpec((block_rows, 1), lambda i: (i, 0)),
            pl.BlockSpec((block_rows, 1), lambda i: (i, 0)),
            pl.BlockSpec((1, d), lambda i: (0, 0)),
        ],
        out_specs=pl.BlockSpec((1, d), lambda i: (0, 0)),
        out_shape=jax.ShapeDtypeStruct((1, d), jnp.float32),
    )(s, h1s, dis, wn, b)


def _edge_reduce_body(z_ref, lam_ref, b_ref, out_ref):
    @pl.when(pl.program_id(0) == 0)
    def _():
        out_ref[...] = jnp.zeros_like(out_ref)

    e = jax.nn.sigmoid(z_ref[...] + b_ref[...])
    out_ref[...] += jnp.dot(lam_ref[...].T, e, preferred_element_type=jnp.float32)


def _edge_reduce(z, lam, b, block_rows):
    n, d = z.shape
    grid = n // block_rows
    return pl.pallas_call(
        _edge_reduce_body,
        grid=(grid,),
        in_specs=[pl.BlockSpec((block_rows, d), lambda i: (i, 0)),
                  pl.BlockSpec((block_rows, 1), lambda i: (i, 0)),
                  pl.BlockSpec((1, d), lambda i: (0, 0))],
        out_specs=pl.BlockSpec((1, d), lambda i: (0, 0)),
        out_shape=jax.ShapeDtypeStruct((1, d), jnp.float32),
    )(z, lam, b)


def _tail_body(ns_ref, es_ref, wn2_ref, we2_ref, wfc_ref, bn2_ref, be2_ref,
               bfc_ref, nn_ref, ee_ref, out_ref):
    node_rep = jnp.dot(ns_ref[...], wn2_ref[...],
                       preferred_element_type=jnp.float32) / nn_ref[0, 0] + bn2_ref[...]
    edge_rep = jnp.dot(es_ref[...], we2_ref[...],
                       preferred_element_type=jnp.float32) / ee_ref[0, 0] + be2_ref[...]
    rep = jnp.concatenate([node_rep, edge_rep], axis=-1)
    out_ref[...] = jnp.dot(rep, wfc_ref[...],
                           preferred_element_type=jnp.float32) + bfc_ref[...]


def _tail(nodesum, edgesum, Wn2, We2, Wfc, bn2, be2, bfc, n_nodes, n_edges):
    nn = jnp.full((1, 1), float(n_nodes), jnp.float32)
    ee = jnp.full((1, 1), float(n_edges), jnp.float32)
    full = lambda shape: pl.BlockSpec(shape, lambda: (0,) * len(shape))
    return pl.pallas_call(
        _tail_body,
        in_specs=[full(nodesum.shape), full(edgesum.shape), full(Wn2.shape),
                  full(We2.shape), full(Wfc.shape), full((1, Wn2.shape[1])),
                  full((1, We2.shape[1])), full((1, Wfc.shape[1])),
                  full((1, 1)), full((1, 1))],
        out_specs=full((1, Wfc.shape[1])),
        out_shape=jax.ShapeDtypeStruct((1, Wfc.shape[1]), jnp.float32),
    )(nodesum, edgesum, Wn2, We2, Wfc, bn2[None, :], be2[None, :], bfc[None, :],
      nn, ee)


def kernel(x, edge_index, edge_attr, batch, Wn1, bn1, Wn2, bn2, We1, be1,
           We2, be2, Wfc, bfc):
    N = x.shape[0]
    E = edge_index.shape[1]
    row, col = edge_index[0], edge_index[1]

    deg = jnp.zeros((N,), jnp.float32).at[col].add(1.0) + 1.0
    dis = deg ** -0.5
    x_p = jnp.pad(x, ((0, _NPAD - N), (0, 0)))
    dis_p = jnp.pad(dis, (0, _NPAD - N))
    h1s_p = _mm_scale(x_p, Wn1, dis_p[:, None], 1024)
    h1s = h1s_p[:N]
    s = _node_scatter_sc(h1s_p, row, col, Wn1.shape[1], E)[:N]
    q = jnp.zeros((N,), jnp.float32).at[row].add(dis[col])
    wnode = dis * q + dis * dis
    nodesum = _node_reduce(s, h1s, dis[:, None], wnode[:, None],
                           bn1[None, :], 1000)

    u0, u1 = row, col
    nodes = edge_index.T.reshape(-1)
    order = jnp.argsort(nodes, stable=True)
    nodes_s = nodes[order]
    idx = jnp.arange(2 * E, dtype=jnp.int32)
    is_start = jnp.concatenate([jnp.array([True]), nodes_s[1:] != nodes_s[:-1]])
    start = jax.lax.cummax(jnp.where(is_start, idx, 0))
    rank = idx - start
    cnt = jnp.zeros((N,), jnp.int32).at[nodes].add(1)
    kcap = jnp.minimum(cnt[nodes_s], 100)
    counted = jnp.zeros((2 * E,), bool).at[order].set(rank < kcap)
    c0, c1 = counted[0::2], counted[1::2]
    selfloop = u0 == u1
    w0 = (c0 | (selfloop & c1)).astype(jnp.float32)
    w1 = (c1 & (~selfloop)).astype(jnp.float32)
    both = c0 & c1
    eligible = both & (~selfloop)
    key = jnp.where(eligible,
                    jnp.minimum(u0, u1) * N + jnp.maximum(u0, u1),
                    N * N)
    p = jnp.argsort(key)
    key_s = key[p]
    gid = jnp.cumsum(jnp.concatenate(
        [jnp.zeros((1,), jnp.int32), (key_s[1:] != key_s[:-1]).astype(jnp.int32)]))
    inv = jnp.argsort(p)
    Fc = jnp.zeros((N,), jnp.float32).at[u0].add(w0).at[u1].add(w1)
    Gc = jax.ops.segment_sum(jnp.ones((E,), jnp.float32), gid,
                             num_segments=E)[gid][inv]
    degA = jnp.where(eligible, Fc[u0] + Fc[u1] - Gc - 1.0,
           jnp.where(both, Fc[u0],
           jnp.where(c0, Fc[u0] - 1.0,
           jnp.where(c1, Fc[u1] - 1.0, 0.0))))
    dise = (degA + 1.0) ** -0.5

    f = _mm_scale(edge_attr, We1, dise[:, None], 8000)
    F = jnp.zeros((N, We1.shape[1]), jnp.float32).at[u0].add(
        f * w0[:, None]).at[u1].add(f * w1[:, None])
    G = jax.ops.segment_sum(f[p], gid, num_segments=E)[gid][inv]

    alpha0 = c0.astype(jnp.float32)
    alpha1 = (eligible | (c1 & ~c0)).astype(jnp.float32)
    alphaf = -((c0 | c1) & ~(both & selfloop)).astype(jnp.float32)
    C = jnp.zeros((N,), jnp.float32).at[u0].add(dise * alpha0).at[u1].add(
        dise * alpha1)
    T = jax.ops.segment_sum(jnp.where(eligible, -dise, 0.0)[p], gid,
                            num_segments=E)[gid][inv]
    lam = dise * (w0 * C[u0] + w1 * C[u1] + T) + dise * dise * (alphaf + 1.0)

    elig_f = eligible.astype(jnp.float32)
    z = (dise * alpha0)[:, None] * F[u0] + (dise * alpha1)[:, None] * F[u1] \
        - (dise * elig_f)[:, None] * G + (dise * (alphaf + 1.0))[:, None] * f
    edgesum = _edge_reduce(z, lam[:, None], be1[None, :], 4000)

    return _tail(nodesum, edgesum, Wn2, We2, Wfc, bn2, be2, bfc, N, E)


# speedup vs baseline: 2.7176x; 1.8176x over previous
import functools

import jax
import jax.numpy as jnp
from jax import lax
from jax.experimental import pallas as pl
from jax.experimental.pallas import tpu as pltpu
from jax.experimental.pallas import tpu_sc as plsc

_NC = 2
_NS = 16
_L = 16


_NPAD = 10240
_HALF = 5120
_REAL_HALF = 5000
_DUMMY = 5100
_EK = 80


def _node_scatter_sc(h1s_p, row, col, d_feat, n_edges):
    epw = n_edges // _NS
    chunks = epw // _EK
    mesh = plsc.VectorSubcoreMesh(core_axis_name="c", subcore_axis_name="s")

    @functools.partial(
        pl.kernel,
        out_type=jax.ShapeDtypeStruct((_NPAD, d_feat), jnp.float32),
        mesh=mesh,
        scratch_types=dict(
            table=pltpu.VMEM_SHARED((_HALF, d_feat), jnp.float32),
            rowflat=pltpu.VMEM((epw,), jnp.int32),
            colflat=pltpu.VMEM((epw,), jnp.int32),
            lidx=pltpu.VMEM((chunks, _EK), jnp.int32),
            rbuf=pltpu.VMEM((_EK, d_feat), jnp.float32),
            zbuf=pltpu.VMEM((_EK, d_feat), jnp.float32),
            sem=pltpu.SemaphoreType.DMA,
        ),
    )
    def k(h1s_hbm, row_hbm, col_hbm, out_hbm, table, rowflat, colflat, lidx,
          rbuf, zbuf, sem):
        cc = lax.axis_index("c")
        tid = lax.axis_index("s")
        base = cc * _REAL_HALF
        nb = d_feat // _L

        def zrow(r, _):
            for j in range(nb):
                zbuf[r, pl.ds(j * _L, _L)] = jnp.zeros((_L,), jnp.float32)
            return 0
        lax.fori_loop(0, _EK, zrow, 0)
        rows_per_tile = _HALF // _NS
        for j in range(rows_per_tile // _EK):
            pltpu.sync_copy(zbuf, table.at[pl.ds(tid * rows_per_tile + j * _EK,
                                                 _EK)])
        plsc.subcore_barrier()

        e0 = tid * epw
        pltpu.sync_copy(row_hbm.at[pl.ds(e0, epw)], rowflat)
        pltpu.sync_copy(col_hbm.at[pl.ds(e0, epw)], colflat)

        def lidx_body(g, _):
            for j in range(_EK // _L):
                cv = colflat[pl.ds(g * _EK + j * _L, _L)]
                lc = cv - base
                m = (lc >= 0) & (lc < _REAL_HALF)
                lidx[g, pl.ds(j * _L, _L)] = jnp.where(m, lc, _DUMMY)
            return 0
        lax.fori_loop(0, chunks, lidx_body, 0)

        def chunk_body(g, _):
            pltpu.async_copy(h1s_hbm.at[rowflat.at[pl.ds(g * _EK, _EK)]],
                             rbuf, sem).wait()
            pltpu.sync_copy(rbuf, table.at[lidx.at[g]], add=True)
            return 0
        lax.fori_loop(0, chunks, chunk_body, 0)
        plsc.subcore_barrier()

        for j in range(rows_per_tile // _EK):
            local = tid * rows_per_tile + j * _EK
            @pl.when(local < _REAL_HALF)
            def _():
                pltpu.sync_copy(table.at[pl.ds(local, _EK)], rbuf)
                pltpu.sync_copy(rbuf, out_hbm.at[pl.ds(base + local, _EK)])

    return k(h1s_p, row, col)


def _mesh():
    return plsc.VectorSubcoreMesh(core_axis_name="c", subcore_axis_name="s")


def _zero1d(ref, n):
    def b(i, _):
        ref[pl.ds(i * _L, _L)] = jnp.zeros((_L,), jnp.float32)
        return 0
    lax.fori_loop(0, n // _L, b, 0)


def _emit_tables(widx, tabs, out_hbm):
    for j, t in enumerate(tabs):
        pltpu.sync_copy(t, out_hbm.at[widx, j])


def _sc_scatter_counts(idxa, idxb, e_cnt):
    ept = e_cnt // (_NC * _NS)
    full, rem = ept // _L, ept % _L
    seg = _NPAD // _NS

    @functools.partial(
        pl.kernel,
        out_type=jax.ShapeDtypeStruct((_NC * _NS, 2, _NPAD), jnp.float32),
        mesh=_mesh(),
        compiler_params=pltpu.CompilerParams(needs_layout_passes=False),
        scratch_types=dict(
            tab0=pltpu.VMEM((_NPAD,), jnp.float32),
            tab1=pltpu.VMEM((_NPAD,), jnp.float32),
            ia=pltpu.VMEM((ept + _L,), jnp.int32),
            ib=pltpu.VMEM((ept + _L,), jnp.int32),
        ),
    )
    def k(ia_hbm, ib_hbm, out_hbm, tab0, tab1, ia, ib):
        cc = lax.axis_index("c")
        tid = lax.axis_index("s")
        e0 = (cc * _NS + tid) * ept
        _zero1d(tab0, _NPAD)
        _zero1d(tab1, _NPAD)
        pltpu.sync_copy(ia_hbm.at[pl.ds(e0, ept)], ia.at[pl.ds(0, ept)])
        pltpu.sync_copy(ib_hbm.at[pl.ds(e0, ept)], ib.at[pl.ds(0, ept)])
        ones = jnp.ones((_L,), jnp.float32)

        def b(g, _):
            plsc.addupdate_scatter(tab0, [ia[pl.ds(g * _L, _L)]], ones)
            plsc.addupdate_scatter(tab1, [ib[pl.ds(g * _L, _L)]], ones)
            return 0
        lax.fori_loop(0, full, b, 0)
        if rem:
            m = lax.iota(jnp.int32, _L) < rem
            plsc.addupdate_scatter(tab0, [ia[pl.ds(full * _L, _L)]], ones, mask=m)
            plsc.addupdate_scatter(tab1, [ib[pl.ds(full * _L, _L)]], ones, mask=m)
        _emit_tables(cc * _NS + tid, [tab0, tab1], out_hbm)

    return k(idxa, idxb)


def _sc_scatter_vals(idxa, idxb, vala, valb, e_cnt, gtab=None):
    ept = e_cnt // (_NC * _NS)
    full, rem = ept // _L, ept % _L
    seg = _NPAD // _NS
    gmode = gtab is not None
    scr = dict(
        tab0=pltpu.VMEM((_NPAD,), jnp.float32),
        ia=pltpu.VMEM((ept + _L,), jnp.int32),
        ib=pltpu.VMEM((ept + _L,), jnp.int32),
    )
    if gmode:
        scr["gbuf"] = pltpu.VMEM((_NPAD,), jnp.float32)
    else:
        scr["va"] = pltpu.VMEM((ept + _L,), jnp.float32)
        scr["vb"] = pltpu.VMEM((ept + _L,), jnp.float32)

    def body(cc, tid, ia_hbm, ib_hbm, va_hbm, vb_hbm, g_hbm, out_hbm,
             tab0, ia, ib, gbuf=None, va=None, vb=None):
        e0 = (cc * _NS + tid) * ept
        _zero1d(tab0, _NPAD)
        pltpu.sync_copy(ia_hbm.at[pl.ds(e0, ept)], ia.at[pl.ds(0, ept)])
        pltpu.sync_copy(ib_hbm.at[pl.ds(e0, ept)], ib.at[pl.ds(0, ept)])
        if gmode:
            pltpu.sync_copy(g_hbm, gbuf)
        else:
            pltpu.sync_copy(va_hbm.at[pl.ds(e0, ept)], va.at[pl.ds(0, ept)])
            pltpu.sync_copy(vb_hbm.at[pl.ds(e0, ept)], vb.at[pl.ds(0, ept)])
        allm = lax.iota(jnp.int32, _L) < _L

        def step(g, m):
            av = ia[pl.ds(g * _L, _L)]
            bv = ib[pl.ds(g * _L, _L)]
            if gmode:
                dv = plsc.load_gather(gbuf, [bv], mask=m)
                plsc.addupdate_scatter(tab0, [av], dv, mask=m)
            else:
                plsc.addupdate_scatter(tab0, [av], va[pl.ds(g * _L, _L)], mask=m)
                plsc.addupdate_scatter(tab0, [bv], vb[pl.ds(g * _L, _L)], mask=m)

        def b(g, _):
            step(g, allm)
            return 0
        lax.fori_loop(0, full, b, 0)
        if rem:
            step(full, lax.iota(jnp.int32, _L) < rem)
        _emit_tables(cc * _NS + tid, [tab0], out_hbm)

    out_type = jax.ShapeDtypeStruct((_NC * _NS, 1, _NPAD), jnp.float32)
    cp = pltpu.CompilerParams(needs_layout_passes=False)
    if gmode:
        @functools.partial(pl.kernel, out_type=out_type, mesh=_mesh(),
                           scratch_types=scr, compiler_params=cp)
        def k(ia_hbm, ib_hbm, g_hbm, out_hbm, tab0, ia, ib, gbuf):
            cc = lax.axis_index("c")
            tid = lax.axis_index("s")
            body(cc, tid, ia_hbm, ib_hbm, None, None, g_hbm, out_hbm,
                 tab0, ia, ib, gbuf=gbuf)
        return k(idxa, idxb, gtab)
    else:
        @functools.partial(pl.kernel, out_type=out_type, mesh=_mesh(),
                           scratch_types=scr, compiler_params=cp)
        def k(ia_hbm, ib_hbm, va_hbm, vb_hbm, out_hbm, tab0, ia, ib, va, vb):
            cc = lax.axis_index("c")
            tid = lax.axis_index("s")
            body(cc, tid, ia_hbm, ib_hbm, va_hbm, vb_hbm, None, out_hbm,
                 tab0, ia, ib, va=va, vb=vb)
        return k(idxa, idxb, vala, valb)


def _sc_gather2(tab, idxa, idxb, e_cnt):
    ept = e_cnt // (_NC * _NS)
    full, rem = ept // _L, ept % _L

    @functools.partial(
        pl.kernel,
        out_type=jax.ShapeDtypeStruct((2 * e_cnt,), jnp.float32),
        mesh=_mesh(),
        compiler_params=pltpu.CompilerParams(needs_layout_passes=False),
        scratch_types=dict(
            tabbuf=pltpu.VMEM((_NPAD,), jnp.float32),
            ia=pltpu.VMEM((ept + _L,), jnp.int32),
            ib=pltpu.VMEM((ept + _L,), jnp.int32),
            oa=pltpu.VMEM((ept + _L,), jnp.float32),
            ob=pltpu.VMEM((ept + _L,), jnp.float32),
        ),
    )
    def k(tab_hbm, ia_hbm, ib_hbm, out_hbm, tabbuf, ia, ib, oa, ob):
        cc = lax.axis_index("c")
        tid = lax.axis_index("s")
        e0 = (cc * _NS + tid) * ept
        pltpu.sync_copy(tab_hbm, tabbuf)
        pltpu.sync_copy(ia_hbm.at[pl.ds(e0, ept)], ia.at[pl.ds(0, ept)])
        pltpu.sync_copy(ib_hbm.at[pl.ds(e0, ept)], ib.at[pl.ds(0, ept)])
        nch = full + (1 if rem else 0)

        def b(g, _):
            m = lax.iota(jnp.int32, _L) + g * _L < ept
            oa[pl.ds(g * _L, _L)] = plsc.load_gather(
                tabbuf, [ia[pl.ds(g * _L, _L)]], mask=m)
            ob[pl.ds(g * _L, _L)] = plsc.load_gather(
                tabbuf, [ib[pl.ds(g * _L, _L)]], mask=m)
            return 0
        lax.fori_loop(0, nch, b, 0)
        pltpu.sync_copy(oa.at[pl.ds(0, ept)], out_hbm.at[pl.ds(e0, ept)])
        pltpu.sync_copy(ob.at[pl.ds(0, ept)],
                        out_hbm.at[pl.ds(e_cnt + e0, ept)])

    return k(tab, idxa, idxb).reshape(2, e_cnt)


def _sc_rank_counted(order, nodes, first_pos, cnt, n_slots):
    spw = n_slots // (_NC * _NS)
    ck = _EK
    chunks = spw // ck

    @functools.partial(
        pl.kernel,
        out_type=jax.ShapeDtypeStruct((n_slots,), jnp.float32),
        mesh=_mesh(),
        compiler_params=pltpu.CompilerParams(needs_layout_passes=False),
        scratch_types=dict(
            fptab=pltpu.VMEM((_NPAD,), jnp.int32),
            cnttab=pltpu.VMEM((_NPAD,), jnp.int32),
            oflat=pltpu.VMEM((spw,), jnp.int32),
            o2d=pltpu.VMEM((chunks, ck), jnp.int32),
            nbuf=pltpu.VMEM((ck,), jnp.int32),
            cbuf=pltpu.VMEM((ck,), jnp.float32),
            sem=pltpu.SemaphoreType.DMA,
        ),
    )
    def k(order_hbm, nodes_hbm, fp_hbm, cnt_hbm, out_hbm, fptab, cnttab,
          oflat, o2d, nbuf, cbuf, sem):
        cc = lax.axis_index("c")
        tid = lax.axis_index("s")
        s0 = (cc * _NS + tid) * spw
        pltpu.sync_copy(fp_hbm, fptab)
        pltpu.sync_copy(cnt_hbm, cnttab)
        pltpu.sync_copy(order_hbm.at[pl.ds(s0, spw)], oflat)

        def stage(g, _):
            pltpu.sync_copy(order_hbm.at[pl.ds(s0 + g * ck, ck)], o2d.at[g])
            return 0
        lax.fori_loop(0, chunks, stage, 0)

        def b(g, _):
            pltpu.async_copy(nodes_hbm.at[oflat.at[pl.ds(g * ck, ck)]],
                             nbuf, sem).wait()
            for j in range(ck // _L):
                nv = nbuf[pl.ds(j * _L, _L)]
                fp = plsc.load_gather(fptab, [nv])
                cv = plsc.load_gather(cnttab, [nv])
                ivec = (s0 + g * ck + j * _L) + lax.iota(jnp.int32, _L)
                ok = (ivec - fp) < jnp.minimum(cv, 100)
                cbuf[pl.ds(j * _L, _L)] = jnp.where(ok, 1.0, 0.0)
            pltpu.sync_copy(cbuf, out_hbm.at[o2d.at[g]])
            return 0
        lax.fori_loop(0, chunks, b, 0)

    return k(order, nodes, first_pos, cnt)


def _sc_f_scatter(rows, t0, t1, d_feat, n_edges):
    epw = n_edges // _NS
    chunks = epw // _EK

    @functools.partial(
        pl.kernel,
        out_type=jax.ShapeDtypeStruct((_NPAD, d_feat), jnp.float32),
        mesh=_mesh(),
        scratch_types=dict(
            table=pltpu.VMEM_SHARED((_HALF, d_feat), jnp.float32),
            t0flat=pltpu.VMEM((epw,), jnp.int32),
            t1flat=pltpu.VMEM((epw,), jnp.int32),
            l0=pltpu.VMEM((chunks, _EK), jnp.int32),
            l1=pltpu.VMEM((chunks, _EK), jnp.int32),
            rbuf=pltpu.VMEM((_EK, d_feat), jnp.float32),
            zbuf=pltpu.VMEM((_EK, d_feat), jnp.float32),
        ),
    )
    def k(rows_hbm, t0_hbm, t1_hbm, out_hbm, table, t0flat, t1flat, l0, l1,
          rbuf, zbuf):
        cc = lax.axis_index("c")
        tid = lax.axis_index("s")
        base = cc * _REAL_HALF
        nb = d_feat // _L

        def zrow(r, _):
            for j in range(nb):
                zbuf[r, pl.ds(j * _L, _L)] = jnp.zeros((_L,), jnp.float32)
            return 0
        lax.fori_loop(0, _EK, zrow, 0)
        rpt = _HALF // _NS
        for j in range(rpt // _EK):
            pltpu.sync_copy(zbuf, table.at[pl.ds(tid * rpt + j * _EK, _EK)])
        plsc.subcore_barrier()

        e0 = tid * epw
        pltpu.sync_copy(t0_hbm.at[pl.ds(e0, epw)], t0flat)
        pltpu.sync_copy(t1_hbm.at[pl.ds(e0, epw)], t1flat)

        def lb(g, _):
            for j in range(_EK // _L):
                for (src, dst) in ((t0flat, l0), (t1flat, l1)):
                    tv = src[pl.ds(g * _EK + j * _L, _L)]
                    lc = tv - base
                    m = (lc >= 0) & (lc < _REAL_HALF)
                    dst[g, pl.ds(j * _L, _L)] = jnp.where(m, lc, _DUMMY)
            return 0
        lax.fori_loop(0, chunks, lb, 0)

        def b(g, _):
            pltpu.sync_copy(rows_hbm.at[pl.ds(e0 + g * _EK, _EK)], rbuf)
            pltpu.sync_copy(rbuf, table.at[l0.at[g]], add=True)
            pltpu.sync_copy(rbuf, table.at[l1.at[g]], add=True)
            return 0
        lax.fori_loop(0, chunks, b, 0)
        plsc.subcore_barrier()

        for j in range(rpt // _EK):
            local = tid * rpt + j * _EK
            @pl.when(local < _REAL_HALF)
            def _():
                pltpu.sync_copy(table.at[pl.ds(local, _EK)], rbuf)
                pltpu.sync_copy(rbuf, out_hbm.at[pl.ds(base + local, _EK)])

    return k(rows, t0, t1)


def _sc_row_gather2(tab_p, idxa, idxb, d_feat, e_cnt):
    ept = e_cnt // (_NC * _NS)
    ck = 40
    chunks = ept // ck

    @functools.partial(
        pl.kernel,
        out_type=jax.ShapeDtypeStruct((2, e_cnt, d_feat), jnp.float32),
        mesh=_mesh(),
        scratch_types=dict(
            ia=pltpu.VMEM((ept,), jnp.int32),
            ib=pltpu.VMEM((ept,), jnp.int32),
            buf=pltpu.VMEM((ck, d_feat), jnp.float32),
            sem=pltpu.SemaphoreType.DMA,
        ),
    )
    def k(tab_hbm, ia_hbm, ib_hbm, out_hbm, ia, ib, buf, sem):
        cc = lax.axis_index("c")
        tid = lax.axis_index("s")
        e0 = (cc * _NS + tid) * ept
        pltpu.sync_copy(ia_hbm.at[pl.ds(e0, ept)], ia)
        pltpu.sync_copy(ib_hbm.at[pl.ds(e0, ept)], ib)

        def b(g, _):
            pltpu.async_copy(tab_hbm.at[ia.at[pl.ds(g * ck, ck)]],
                             buf, sem).wait()
            pltpu.sync_copy(buf, out_hbm.at[0, pl.ds(e0 + g * ck, ck)])
            pltpu.async_copy(tab_hbm.at[ib.at[pl.ds(g * ck, ck)]],
                             buf, sem).wait()
            pltpu.sync_copy(buf, out_hbm.at[1, pl.ds(e0 + g * ck, ck)])
            return 0
        lax.fori_loop(0, chunks, b, 0)

    return k(tab_p, idxa, idxb)


def _mm_scale_body(x_ref, w_ref, scale_ref, out_ref):
    out_ref[...] = scale_ref[...] * jnp.dot(
        x_ref[...], w_ref[...], preferred_element_type=jnp.float32)


def _mm_scale(x, w, scale, block_rows):
    n, k = x.shape
    m = w.shape[1]
    grid = n // block_rows
    return pl.pallas_call(
        _mm_scale_body,
        grid=(grid,),
        in_specs=[
            pl.BlockSpec((block_rows, k), lambda i: (i, 0)),
            pl.BlockSpec((k, m), lambda i: (0, 0)),
            pl.BlockSpec((block_rows, 1), lambda i: (i, 0)),
        ],
        out_specs=pl.BlockSpec((block_rows, m), lambda i: (i, 0)),
        out_shape=jax.ShapeDtypeStruct((n, m), jnp.float32),
    )(x, w, scale)


def _node_reduce_body(s_ref, h1s_ref, dis_ref, wn_ref, b_ref, out_ref):
    @pl.when(pl.program_id(0) == 0)
    def _():
        out_ref[...] = jnp.zeros_like(out_ref)

    h = jax.nn.sigmoid(dis_ref[...] * (s_ref[...] + h1s_ref[...]) + b_ref[...])
    out_ref[...] += jnp.dot(wn_ref[...].T, h, preferred_element_type=jnp.float32)


def _node_reduce(s, h1s, dis, wn, b, block_rows):
    n, d = s.shape
    grid = n // block_rows
    return pl.pallas_call(
        _node_reduce_body,
        grid=(grid,),
        in_specs=[
            pl.BlockSpec((block_rows, d), lambda i: (i, 0)),
            pl.BlockSpec((block_rows, d), lambda i: (i, 0)),
            pl.BlockSpec((block_rows, 1), lambda i: (i, 0)),
            pl.BlockSpec((block_rows, 1), lambda i: (i, 0)),
            pl.BlockSpec((1, d), lambda i: (0, 0)),
        ],
        out_specs=pl.BlockSpec((1, d), lambda i: (0, 0)),
        out_shape=jax.ShapeDtypeStruct((1, d), jnp.float32),
    )(s, h1s, dis, wn, b)


def _edge_reduce_body(z_ref, lam_ref, b_ref, out_ref):
    @pl.when(pl.program_id(0) == 0)
    def _():
        out_ref[...] = jnp.zeros_like(out_ref)

    e = jax.nn.sigmoid(z_ref[...] + b_ref[...])
    out_ref[...] += jnp.dot(lam_ref[...].T, e, preferred_element_type=jnp.float32)


def _edge_reduce(z, lam, b, block_rows):
    n, d = z.shape
    grid = n // block_rows
    return pl.pallas_call(
        _edge_reduce_body,
        grid=(grid,),
        in_specs=[pl.BlockSpec((block_rows, d), lambda i: (i, 0)),
                  pl.BlockSpec((block_rows, 1), lambda i: (i, 0)),
                  pl.BlockSpec((1, d), lambda i: (0, 0))],
        out_specs=pl.BlockSpec((1, d), lambda i: (0, 0)),
        out_shape=jax.ShapeDtypeStruct((1, d), jnp.float32),
    )(z, lam, b)


def _tail_body(ns_ref, es_ref, wn2_ref, we2_ref, wfc_ref, bn2_ref, be2_ref,
               bfc_ref, nn_ref, ee_ref, out_ref):
    node_rep = jnp.dot(ns_ref[...], wn2_ref[...],
                       preferred_element_type=jnp.float32) / nn_ref[0, 0] + bn2_ref[...]
    edge_rep = jnp.dot(es_ref[...], we2_ref[...],
                       preferred_element_type=jnp.float32) / ee_ref[0, 0] + be2_ref[...]
    rep = jnp.concatenate([node_rep, edge_rep], axis=-1)
    out_ref[...] = jnp.dot(rep, wfc_ref[...],
                           preferred_element_type=jnp.float32) + bfc_ref[...]


def _tail(nodesum, edgesum, Wn2, We2, Wfc, bn2, be2, bfc, n_nodes, n_edges):
    nn = jnp.full((1, 1), float(n_nodes), jnp.float32)
    ee = jnp.full((1, 1), float(n_edges), jnp.float32)
    full = lambda shape: pl.BlockSpec(shape, lambda: (0,) * len(shape))
    return pl.pallas_call(
        _tail_body,
        in_specs=[full(nodesum.shape), full(edgesum.shape), full(Wn2.shape),
                  full(We2.shape), full(Wfc.shape), full((1, Wn2.shape[1])),
                  full((1, We2.shape[1])), full((1, Wfc.shape[1])),
                  full((1, 1)), full((1, 1))],
        out_specs=full((1, Wfc.shape[1])),
        out_shape=jax.ShapeDtypeStruct((1, Wfc.shape[1]), jnp.float32),
    )(nodesum, edgesum, Wn2, We2, Wfc, bn2[None, :], be2[None, :], bfc[None, :],
      nn, ee)


def kernel(x, edge_index, edge_attr, batch, Wn1, bn1, Wn2, bn2, We1, be1,
           We2, be2, Wfc, bfc):
    N = x.shape[0]
    E = edge_index.shape[1]
    row, col = edge_index[0], edge_index[1]

    ab = _sc_scatter_counts(row, col, E)
    cnt_row = jnp.sum(ab[:, 0], axis=0)
    cnt_col = jnp.sum(ab[:, 1], axis=0)
    deg = cnt_col[:N] + 1.0
    dis = deg ** -0.5
    dis_p = jnp.pad(dis, (0, _NPAD - N))

    x_p = jnp.pad(x, ((0, _NPAD - N), (0, 0)))
    h1s_p = _mm_scale(x_p, Wn1, dis_p[:, None], 1024)
    h1s = h1s_p[:N]
    s = _node_scatter_sc(h1s_p, row, col, Wn1.shape[1], E)[:N]
    q = _sc_scatter_vals(row, col, None, None, E, gtab=dis_p)
    wnode = dis * jnp.sum(q[:, 0], axis=0)[:N] + dis * dis
    nodesum = _node_reduce(s, h1s, dis[:, None], wnode[:, None],
                           bn1[None, :], 1000)

    u0, u1 = row, col
    nodes = edge_index.T.reshape(-1)
    order = jnp.argsort(nodes, stable=True).astype(jnp.int32)
    cnt = (cnt_row + cnt_col).astype(jnp.int32)
    first_pos = jnp.concatenate(
        [jnp.zeros((1,), jnp.int32), jnp.cumsum(cnt)[:-1].astype(jnp.int32)])
    counted_f = _sc_rank_counted(order, nodes, first_pos, cnt, 2 * E)
    counted = counted_f > 0.5
    c0, c1 = counted[0::2], counted[1::2]
    selfloop = u0 == u1
    w0 = (c0 | (selfloop & c1)).astype(jnp.float32)
    w1 = (c1 & (~selfloop)).astype(jnp.float32)
    both = c0 & c1
    eligible = both & (~selfloop)
    key = jnp.where(eligible,
                    jnp.minimum(u0, u1) * N + jnp.maximum(u0, u1),
                    N * N)
    p = jnp.argsort(key).astype(jnp.int32)
    key_s = key[p]
    gid = jnp.cumsum(jnp.concatenate(
        [jnp.zeros((1,), jnp.int32), (key_s[1:] != key_s[:-1]).astype(jnp.int32)]))
    fc2 = _sc_scatter_vals(u0, u1, w0, w1, E)
    Fc = jnp.sum(fc2[:, 0], axis=0)
    fcg = _sc_gather2(Fc, u0, u1, E)
    Fcu0, Fcu1 = fcg[0], fcg[1]
    Gc_s = jax.ops.segment_sum(jnp.ones((E,), jnp.float32), gid,
                               num_segments=E)[gid]
    Gc = jnp.zeros((E,), jnp.float32).at[p].set(Gc_s)
    degA = jnp.where(eligible, Fcu0 + Fcu1 - Gc - 1.0,
           jnp.where(both, Fcu0,
           jnp.where(c0, Fcu0 - 1.0,
           jnp.where(c1, Fcu1 - 1.0, 0.0))))
    dise = (degA + 1.0) ** -0.5

    We1_p = jnp.pad(We1, ((0, 0), (0, 128 - We1.shape[1])))
    f_pad = _mm_scale(edge_attr, We1_p, dise[:, None], 8000)
    f = f_pad[:, :We1.shape[1]]
    big = 4 * _NPAD
    t0 = jnp.where(w0 > 0, u0, big).astype(jnp.int32)
    t1 = jnp.where(w1 > 0, u1, big).astype(jnp.int32)
    F_p = _sc_f_scatter(f_pad, t0, t1, 128, E)
    fg = _sc_row_gather2(F_p, u0, u1, 128, E)
    Fu0 = fg[0, :, :We1.shape[1]]
    Fu1 = fg[1, :, :We1.shape[1]]
    G_s = jax.ops.segment_sum(f[p], gid, num_segments=E)[gid]
    G = jnp.zeros((E, We1.shape[1]), jnp.float32).at[p].set(G_s)

    alpha0 = c0.astype(jnp.float32)
    alpha1 = (eligible | (c1 & ~c0)).astype(jnp.float32)
    alphaf = -((c0 | c1) & ~(both & selfloop)).astype(jnp.float32)
    cv2 = _sc_scatter_vals(u0, u1, dise * alpha0, dise * alpha1, E)
    C = jnp.sum(cv2[:, 0], axis=0)
    cg = _sc_gather2(C, u0, u1, E)
    T_s = jax.ops.segment_sum(jnp.where(eligible, -dise, 0.0)[p], gid,
                              num_segments=E)[gid]
    T = jnp.zeros((E,), jnp.float32).at[p].set(T_s)
    lam = dise * (w0 * cg[0] + w1 * cg[1] + T) + dise * dise * (alphaf + 1.0)

    elig_f = eligible.astype(jnp.float32)
    z = (dise * alpha0)[:, None] * Fu0 + (dise * alpha1)[:, None] * Fu1 \
        - (dise * elig_f)[:, None] * G + (dise * (alphaf + 1.0))[:, None] * f
    edgesum = _edge_reduce(z, lam[:, None], be1[None, :], 4000)

    return _tail(nodesum, edgesum, Wn2, We2, Wfc, bn2, be2, bfc, N, E)


# speedup vs baseline: 3.6112x; 1.3288x over previous
import functools

import jax
import jax.numpy as jnp
from jax import lax
from jax.experimental import pallas as pl
from jax.experimental.pallas import tpu as pltpu
from jax.experimental.pallas import tpu_sc as plsc

_NC = 2
_NS = 16
_L = 16


_NPAD = 10240
_HALF = 5120
_REAL_HALF = 5000
_DUMMY = 5100
_EK = 80


def _node_scatter_sc(h1s_p, row, col, d_feat, n_edges):
    epw = n_edges // _NS
    chunks = epw // _EK
    mesh = plsc.VectorSubcoreMesh(core_axis_name="c", subcore_axis_name="s")

    @functools.partial(
        pl.kernel,
        out_type=jax.ShapeDtypeStruct((_NPAD, d_feat), jnp.float32),
        mesh=mesh,
        scratch_types=dict(
            table=pltpu.VMEM_SHARED((_HALF, d_feat), jnp.float32),
            rowflat=pltpu.VMEM((epw,), jnp.int32),
            colflat=pltpu.VMEM((epw,), jnp.int32),
            lidx=pltpu.VMEM((chunks, _EK), jnp.int32),
            rbuf=pltpu.VMEM((_EK, d_feat), jnp.float32),
            zbuf=pltpu.VMEM((_EK, d_feat), jnp.float32),
            sem=pltpu.SemaphoreType.DMA,
        ),
    )
    def k(h1s_hbm, row_hbm, col_hbm, out_hbm, table, rowflat, colflat, lidx,
          rbuf, zbuf, sem):
        cc = lax.axis_index("c")
        tid = lax.axis_index("s")
        base = cc * _REAL_HALF
        nb = d_feat // _L

        def zrow(r, _):
            for j in range(nb):
                zbuf[r, pl.ds(j * _L, _L)] = jnp.zeros((_L,), jnp.float32)
            return 0
        lax.fori_loop(0, _EK, zrow, 0)
        rows_per_tile = _HALF // _NS
        for j in range(rows_per_tile // _EK):
            pltpu.sync_copy(zbuf, table.at[pl.ds(tid * rows_per_tile + j * _EK,
                                                 _EK)])
        plsc.subcore_barrier()

        e0 = tid * epw
        pltpu.sync_copy(row_hbm.at[pl.ds(e0, epw)], rowflat)
        pltpu.sync_copy(col_hbm.at[pl.ds(e0, epw)], colflat)

        def lidx_body(g, _):
            for j in range(_EK // _L):
                cv = colflat[pl.ds(g * _EK + j * _L, _L)]
                lc = cv - base
                m = (lc >= 0) & (lc < _REAL_HALF)
                lidx[g, pl.ds(j * _L, _L)] = jnp.where(m, lc, _DUMMY)
            return 0
        lax.fori_loop(0, chunks, lidx_body, 0)

        def chunk_body(g, _):
            pltpu.async_copy(h1s_hbm.at[rowflat.at[pl.ds(g * _EK, _EK)]],
                             rbuf, sem).wait()
            pltpu.sync_copy(rbuf, table.at[lidx.at[g]], add=True)
            return 0
        lax.fori_loop(0, chunks, chunk_body, 0)
        plsc.subcore_barrier()

        for j in range(rows_per_tile // _EK):
            local = tid * rows_per_tile + j * _EK
            @pl.when(local < _REAL_HALF)
            def _():
                pltpu.sync_copy(table.at[pl.ds(local, _EK)], rbuf)
                pltpu.sync_copy(rbuf, out_hbm.at[pl.ds(base + local, _EK)])

    return k(h1s_p, row, col)


def _mesh():
    return plsc.VectorSubcoreMesh(core_axis_name="c", subcore_axis_name="s")


def _zero1d(ref, n):
    def b(i, _):
        ref[pl.ds(i * _L, _L)] = jnp.zeros((_L,), jnp.float32)
        return 0
    lax.fori_loop(0, n // _L, b, 0)


def _emit_tables(widx, tabs, out_hbm):
    for j, t in enumerate(tabs):
        pltpu.sync_copy(t, out_hbm.at[widx, j])


def _sc_scatter_counts(idxa, idxb, e_cnt):
    ept = e_cnt // (_NC * _NS)
    full, rem = ept // _L, ept % _L
    seg = _NPAD // _NS

    @functools.partial(
        pl.kernel,
        out_type=jax.ShapeDtypeStruct((_NC * _NS, 2, _NPAD), jnp.float32),
        mesh=_mesh(),
        compiler_params=pltpu.CompilerParams(needs_layout_passes=False),
        scratch_types=dict(
            tab0=pltpu.VMEM((_NPAD,), jnp.float32),
            tab1=pltpu.VMEM((_NPAD,), jnp.float32),
            ia=pltpu.VMEM((ept + _L,), jnp.int32),
            ib=pltpu.VMEM((ept + _L,), jnp.int32),
        ),
    )
    def k(ia_hbm, ib_hbm, out_hbm, tab0, tab1, ia, ib):
        cc = lax.axis_index("c")
        tid = lax.axis_index("s")
        e0 = (cc * _NS + tid) * ept
        _zero1d(tab0, _NPAD)
        _zero1d(tab1, _NPAD)
        pltpu.sync_copy(ia_hbm.at[pl.ds(e0, ept)], ia.at[pl.ds(0, ept)])
        pltpu.sync_copy(ib_hbm.at[pl.ds(e0, ept)], ib.at[pl.ds(0, ept)])
        ones = jnp.ones((_L,), jnp.float32)

        def b(g, _):
            plsc.addupdate_scatter(tab0, [ia[pl.ds(g * _L, _L)]], ones)
            plsc.addupdate_scatter(tab1, [ib[pl.ds(g * _L, _L)]], ones)
            return 0
        lax.fori_loop(0, full, b, 0)
        if rem:
            m = lax.iota(jnp.int32, _L) < rem
            plsc.addupdate_scatter(tab0, [ia[pl.ds(full * _L, _L)]], ones, mask=m)
            plsc.addupdate_scatter(tab1, [ib[pl.ds(full * _L, _L)]], ones, mask=m)
        _emit_tables(cc * _NS + tid, [tab0, tab1], out_hbm)

    return k(idxa, idxb)


def _sc_scatter_vals(idxa, idxb, vala, valb, e_cnt, gtab=None):
    ept = e_cnt // (_NC * _NS)
    full, rem = ept // _L, ept % _L
    seg = _NPAD // _NS
    gmode = gtab is not None
    scr = dict(
        tab0=pltpu.VMEM((_NPAD,), jnp.float32),
        ia=pltpu.VMEM((ept + _L,), jnp.int32),
        ib=pltpu.VMEM((ept + _L,), jnp.int32),
    )
    if gmode:
        scr["gbuf"] = pltpu.VMEM((_NPAD,), jnp.float32)
    else:
        scr["va"] = pltpu.VMEM((ept + _L,), jnp.float32)
        scr["vb"] = pltpu.VMEM((ept + _L,), jnp.float32)

    def body(cc, tid, ia_hbm, ib_hbm, va_hbm, vb_hbm, g_hbm, out_hbm,
             tab0, ia, ib, gbuf=None, va=None, vb=None):
        e0 = (cc * _NS + tid) * ept
        _zero1d(tab0, _NPAD)
        pltpu.sync_copy(ia_hbm.at[pl.ds(e0, ept)], ia.at[pl.ds(0, ept)])
        pltpu.sync_copy(ib_hbm.at[pl.ds(e0, ept)], ib.at[pl.ds(0, ept)])
        if gmode:
            pltpu.sync_copy(g_hbm, gbuf)
        else:
            pltpu.sync_copy(va_hbm.at[pl.ds(e0, ept)], va.at[pl.ds(0, ept)])
            pltpu.sync_copy(vb_hbm.at[pl.ds(e0, ept)], vb.at[pl.ds(0, ept)])
        allm = lax.iota(jnp.int32, _L) < _L

        def step(g, m):
            av = ia[pl.ds(g * _L, _L)]
            bv = ib[pl.ds(g * _L, _L)]
            if gmode:
                dv = plsc.load_gather(gbuf, [bv], mask=m)
                plsc.addupdate_scatter(tab0, [av], dv, mask=m)
            else:
                plsc.addupdate_scatter(tab0, [av], va[pl.ds(g * _L, _L)], mask=m)
                plsc.addupdate_scatter(tab0, [bv], vb[pl.ds(g * _L, _L)], mask=m)

        def b(g, _):
            step(g, allm)
            return 0
        lax.fori_loop(0, full, b, 0)
        if rem:
            step(full, lax.iota(jnp.int32, _L) < rem)
        _emit_tables(cc * _NS + tid, [tab0], out_hbm)

    out_type = jax.ShapeDtypeStruct((_NC * _NS, 1, _NPAD), jnp.float32)
    cp = pltpu.CompilerParams(needs_layout_passes=False)
    if gmode:
        @functools.partial(pl.kernel, out_type=out_type, mesh=_mesh(),
                           scratch_types=scr, compiler_params=cp)
        def k(ia_hbm, ib_hbm, g_hbm, out_hbm, tab0, ia, ib, gbuf):
            cc = lax.axis_index("c")
            tid = lax.axis_index("s")
            body(cc, tid, ia_hbm, ib_hbm, None, None, g_hbm, out_hbm,
                 tab0, ia, ib, gbuf=gbuf)
        return k(idxa, idxb, gtab)
    else:
        @functools.partial(pl.kernel, out_type=out_type, mesh=_mesh(),
                           scratch_types=scr, compiler_params=cp)
        def k(ia_hbm, ib_hbm, va_hbm, vb_hbm, out_hbm, tab0, ia, ib, va, vb):
            cc = lax.axis_index("c")
            tid = lax.axis_index("s")
            body(cc, tid, ia_hbm, ib_hbm, va_hbm, vb_hbm, None, out_hbm,
                 tab0, ia, ib, va=va, vb=vb)
        return k(idxa, idxb, vala, valb)


def _sc_gather2(tab, idxa, idxb, e_cnt):
    ept = e_cnt // (_NC * _NS)
    full, rem = ept // _L, ept % _L

    @functools.partial(
        pl.kernel,
        out_type=jax.ShapeDtypeStruct((2 * e_cnt,), jnp.float32),
        mesh=_mesh(),
        compiler_params=pltpu.CompilerParams(needs_layout_passes=False),
        scratch_types=dict(
            tabbuf=pltpu.VMEM((_NPAD,), jnp.float32),
            ia=pltpu.VMEM((ept + _L,), jnp.int32),
            ib=pltpu.VMEM((ept + _L,), jnp.int32),
            oa=pltpu.VMEM((ept + _L,), jnp.float32),
            ob=pltpu.VMEM((ept + _L,), jnp.float32),
        ),
    )
    def k(tab_hbm, ia_hbm, ib_hbm, out_hbm, tabbuf, ia, ib, oa, ob):
        cc = lax.axis_index("c")
        tid = lax.axis_index("s")
        e0 = (cc * _NS + tid) * ept
        pltpu.sync_copy(tab_hbm, tabbuf)
        pltpu.sync_copy(ia_hbm.at[pl.ds(e0, ept)], ia.at[pl.ds(0, ept)])
        pltpu.sync_copy(ib_hbm.at[pl.ds(e0, ept)], ib.at[pl.ds(0, ept)])
        nch = full + (1 if rem else 0)

        def b(g, _):
            m = lax.iota(jnp.int32, _L) + g * _L < ept
            oa[pl.ds(g * _L, _L)] = plsc.load_gather(
                tabbuf, [ia[pl.ds(g * _L, _L)]], mask=m)
            ob[pl.ds(g * _L, _L)] = plsc.load_gather(
                tabbuf, [ib[pl.ds(g * _L, _L)]], mask=m)
            return 0
        lax.fori_loop(0, nch, b, 0)
        pltpu.sync_copy(oa.at[pl.ds(0, ept)], out_hbm.at[pl.ds(e0, ept)])
        pltpu.sync_copy(ob.at[pl.ds(0, ept)],
                        out_hbm.at[pl.ds(e_cnt + e0, ept)])

    return k(tab, idxa, idxb).reshape(2, e_cnt)


def _sc_rank_counted(order, nodes, first_pos, cnt, n_slots):
    spw = n_slots // (_NC * _NS)
    ck = _EK
    chunks = spw // ck

    @functools.partial(
        pl.kernel,
        out_type=jax.ShapeDtypeStruct((n_slots,), jnp.float32),
        mesh=_mesh(),
        compiler_params=pltpu.CompilerParams(needs_layout_passes=False),
        scratch_types=dict(
            fptab=pltpu.VMEM((_NPAD,), jnp.int32),
            cnttab=pltpu.VMEM((_NPAD,), jnp.int32),
            oflat=pltpu.VMEM((spw,), jnp.int32),
            o2d=pltpu.VMEM((chunks, ck), jnp.int32),
            nbuf=pltpu.VMEM((ck,), jnp.int32),
            cbuf=pltpu.VMEM((ck,), jnp.float32),
            sem=pltpu.SemaphoreType.DMA,
        ),
    )
    def k(order_hbm, nodes_hbm, fp_hbm, cnt_hbm, out_hbm, fptab, cnttab,
          oflat, o2d, nbuf, cbuf, sem):
        cc = lax.axis_index("c")
        tid = lax.axis_index("s")
        s0 = (cc * _NS + tid) * spw
        pltpu.sync_copy(fp_hbm, fptab)
        pltpu.sync_copy(cnt_hbm, cnttab)
        pltpu.sync_copy(order_hbm.at[pl.ds(s0, spw)], oflat)

        def stage(g, _):
            pltpu.sync_copy(order_hbm.at[pl.ds(s0 + g * ck, ck)], o2d.at[g])
            return 0
        lax.fori_loop(0, chunks, stage, 0)

        def b(g, _):
            pltpu.async_copy(nodes_hbm.at[oflat.at[pl.ds(g * ck, ck)]],
                             nbuf, sem).wait()
            for j in range(ck // _L):
                nv = nbuf[pl.ds(j * _L, _L)]
                fp = plsc.load_gather(fptab, [nv])
                cv = plsc.load_gather(cnttab, [nv])
                ivec = (s0 + g * ck + j * _L) + lax.iota(jnp.int32, _L)
                ok = (ivec - fp) < jnp.minimum(cv, 100)
                cbuf[pl.ds(j * _L, _L)] = jnp.where(ok, 1.0, 0.0)
            pltpu.sync_copy(cbuf, out_hbm.at[o2d.at[g]])
            return 0
        lax.fori_loop(0, chunks, b, 0)

    return k(order, nodes, first_pos, cnt)


def _sc_f_scatter(rows, t0, t1, d_feat, n_edges):
    epw = n_edges // _NS
    chunks = epw // _EK

    @functools.partial(
        pl.kernel,
        out_type=jax.ShapeDtypeStruct((_NPAD, d_feat), jnp.float32),
        mesh=_mesh(),
        scratch_types=dict(
            table=pltpu.VMEM_SHARED((_HALF, d_feat), jnp.float32),
            t0flat=pltpu.VMEM((epw,), jnp.int32),
            t1flat=pltpu.VMEM((epw,), jnp.int32),
            l0=pltpu.VMEM((chunks, _EK), jnp.int32),
            l1=pltpu.VMEM((chunks, _EK), jnp.int32),
            rbuf=pltpu.VMEM((_EK, d_feat), jnp.float32),
            zbuf=pltpu.VMEM((_EK, d_feat), jnp.float32),
        ),
    )
    def k(rows_hbm, t0_hbm, t1_hbm, out_hbm, table, t0flat, t1flat, l0, l1,
          rbuf, zbuf):
        cc = lax.axis_index("c")
        tid = lax.axis_index("s")
        base = cc * _REAL_HALF
        nb = d_feat // _L

        def zrow(r, _):
            for j in range(nb):
                zbuf[r, pl.ds(j * _L, _L)] = jnp.zeros((_L,), jnp.float32)
            return 0
        lax.fori_loop(0, _EK, zrow, 0)
        rpt = _HALF // _NS
        for j in range(rpt // _EK):
            pltpu.sync_copy(zbuf, table.at[pl.ds(tid * rpt + j * _EK, _EK)])
        plsc.subcore_barrier()

        e0 = tid * epw
        pltpu.sync_copy(t0_hbm.at[pl.ds(e0, epw)], t0flat)
        pltpu.sync_copy(t1_hbm.at[pl.ds(e0, epw)], t1flat)

        def lb(g, _):
            for j in range(_EK // _L):
                for (src, dst) in ((t0flat, l0), (t1flat, l1)):
                    tv = src[pl.ds(g * _EK + j * _L, _L)]
                    lc = tv - base
                    m = (lc >= 0) & (lc < _REAL_HALF)
                    dst[g, pl.ds(j * _L, _L)] = jnp.where(m, lc, _DUMMY)
            return 0
        lax.fori_loop(0, chunks, lb, 0)

        def b(g, _):
            pltpu.sync_copy(rows_hbm.at[pl.ds(e0 + g * _EK, _EK)], rbuf)
            pltpu.sync_copy(rbuf, table.at[l0.at[g]], add=True)
            pltpu.sync_copy(rbuf, table.at[l1.at[g]], add=True)
            return 0
        lax.fori_loop(0, chunks, b, 0)
        plsc.subcore_barrier()

        for j in range(rpt // _EK):
            local = tid * rpt + j * _EK
            @pl.when(local < _REAL_HALF)
            def _():
                pltpu.sync_copy(table.at[pl.ds(local, _EK)], rbuf)
                pltpu.sync_copy(rbuf, out_hbm.at[pl.ds(base + local, _EK)])

    return k(rows, t0, t1)


def _sc_row_gather2(tab_p, idxa, idxb, d_feat, e_cnt):
    ept = e_cnt // (_NC * _NS)
    ck = 40
    chunks = ept // ck

    @functools.partial(
        pl.kernel,
        out_type=jax.ShapeDtypeStruct((2, e_cnt, d_feat), jnp.float32),
        mesh=_mesh(),
        scratch_types=dict(
            ia=pltpu.VMEM((ept,), jnp.int32),
            ib=pltpu.VMEM((ept,), jnp.int32),
            buf=pltpu.VMEM((ck, d_feat), jnp.float32),
            sem=pltpu.SemaphoreType.DMA,
        ),
    )
    def k(tab_hbm, ia_hbm, ib_hbm, out_hbm, ia, ib, buf, sem):
        cc = lax.axis_index("c")
        tid = lax.axis_index("s")
        e0 = (cc * _NS + tid) * ept
        pltpu.sync_copy(ia_hbm.at[pl.ds(e0, ept)], ia)
        pltpu.sync_copy(ib_hbm.at[pl.ds(e0, ept)], ib)

        def b(g, _):
            pltpu.async_copy(tab_hbm.at[ia.at[pl.ds(g * ck, ck)]],
                             buf, sem).wait()
            pltpu.sync_copy(buf, out_hbm.at[0, pl.ds(e0 + g * ck, ck)])
            pltpu.async_copy(tab_hbm.at[ib.at[pl.ds(g * ck, ck)]],
                             buf, sem).wait()
            pltpu.sync_copy(buf, out_hbm.at[1, pl.ds(e0 + g * ck, ck)])
            return 0
        lax.fori_loop(0, chunks, b, 0)

    return k(tab_p, idxa, idxb)


def _mm_scale_body(x_ref, w_ref, scale_ref, out_ref):
    out_ref[...] = scale_ref[...] * jnp.dot(
        x_ref[...], w_ref[...], preferred_element_type=jnp.float32)


def _mm_scale(x, w, scale, block_rows):
    n, k = x.shape
    m = w.shape[1]
    grid = n // block_rows
    return pl.pallas_call(
        _mm_scale_body,
        grid=(grid,),
        in_specs=[
            pl.BlockSpec((block_rows, k), lambda i: (i, 0)),
            pl.BlockSpec((k, m), lambda i: (0, 0)),
            pl.BlockSpec((block_rows, 1), lambda i: (i, 0)),
        ],
        out_specs=pl.BlockSpec((block_rows, m), lambda i: (i, 0)),
        out_shape=jax.ShapeDtypeStruct((n, m), jnp.float32),
    )(x, w, scale)


def _node_reduce_body(s_ref, h1s_ref, dis_ref, wn_ref, b_ref, out_ref):
    @pl.when(pl.program_id(0) == 0)
    def _():
        out_ref[...] = jnp.zeros_like(out_ref)

    h = jax.nn.sigmoid(dis_ref[...] * (s_ref[...] + h1s_ref[...]) + b_ref[...])
    out_ref[...] += jnp.dot(wn_ref[...].T, h, preferred_element_type=jnp.float32)


def _node_reduce(s, h1s, dis, wn, b, block_rows):
    n, d = s.shape
    grid = n // block_rows
    return pl.pallas_call(
        _node_reduce_body,
        grid=(grid,),
        in_specs=[
            pl.BlockSpec((block_rows, d), lambda i: (i, 0)),
            pl.BlockSpec((block_rows, d), lambda i: (i, 0)),
            pl.BlockSpec((block_rows, 1), lambda i: (i, 0)),
            pl.BlockSpec((block_rows, 1), lambda i: (i, 0)),
            pl.BlockSpec((1, d), lambda i: (0, 0)),
        ],
        out_specs=pl.BlockSpec((1, d), lambda i: (0, 0)),
        out_shape=jax.ShapeDtypeStruct((1, d), jnp.float32),
    )(s, h1s, dis, wn, b)


def _edge_reduce_body(z_ref, lam_ref, b_ref, out_ref):
    @pl.when(pl.program_id(0) == 0)
    def _():
        out_ref[...] = jnp.zeros_like(out_ref)

    e = jax.nn.sigmoid(z_ref[...] + b_ref[...])
    out_ref[...] += jnp.dot(lam_ref[...].T, e, preferred_element_type=jnp.float32)


def _edge_reduce(z, lam, b, block_rows):
    n, d = z.shape
    grid = n // block_rows
    return pl.pallas_call(
        _edge_reduce_body,
        grid=(grid,),
        in_specs=[pl.BlockSpec((block_rows, d), lambda i: (i, 0)),
                  pl.BlockSpec((block_rows, 1), lambda i: (i, 0)),
                  pl.BlockSpec((1, d), lambda i: (0, 0))],
        out_specs=pl.BlockSpec((1, d), lambda i: (0, 0)),
        out_shape=jax.ShapeDtypeStruct((1, d), jnp.float32),
    )(z, lam, b)


def _tail_body(ns_ref, es_ref, wn2_ref, we2_ref, wfc_ref, bn2_ref, be2_ref,
               bfc_ref, nn_ref, ee_ref, out_ref):
    node_rep = jnp.dot(ns_ref[...], wn2_ref[...],
                       preferred_element_type=jnp.float32) / nn_ref[0, 0] + bn2_ref[...]
    edge_rep = jnp.dot(es_ref[...], we2_ref[...],
                       preferred_element_type=jnp.float32) / ee_ref[0, 0] + be2_ref[...]
    rep = jnp.concatenate([node_rep, edge_rep], axis=-1)
    out_ref[...] = jnp.dot(rep, wfc_ref[...],
                           preferred_element_type=jnp.float32) + bfc_ref[...]


def _tail(nodesum, edgesum, Wn2, We2, Wfc, bn2, be2, bfc, n_nodes, n_edges):
    nn = jnp.full((1, 1), float(n_nodes), jnp.float32)
    ee = jnp.full((1, 1), float(n_edges), jnp.float32)
    full = lambda shape: pl.BlockSpec(shape, lambda: (0,) * len(shape))
    return pl.pallas_call(
        _tail_body,
        in_specs=[full(nodesum.shape), full(edgesum.shape), full(Wn2.shape),
                  full(We2.shape), full(Wfc.shape), full((1, Wn2.shape[1])),
                  full((1, We2.shape[1])), full((1, Wfc.shape[1])),
                  full((1, 1)), full((1, 1))],
        out_specs=full((1, Wfc.shape[1])),
        out_shape=jax.ShapeDtypeStruct((1, Wfc.shape[1]), jnp.float32),
    )(nodesum, edgesum, Wn2, We2, Wfc, bn2[None, :], be2[None, :], bfc[None, :],
      nn, ee)


def kernel(x, edge_index, edge_attr, batch, Wn1, bn1, Wn2, bn2, We1, be1,
           We2, be2, Wfc, bfc):
    N = x.shape[0]
    E = edge_index.shape[1]
    row, col = edge_index[0], edge_index[1]

    ab = _sc_scatter_counts(row, col, E)
    cnt_row = jnp.sum(ab[:, 0], axis=0)
    cnt_col = jnp.sum(ab[:, 1], axis=0)
    deg = cnt_col[:N] + 1.0
    dis = deg ** -0.5
    dis_p = jnp.pad(dis, (0, _NPAD - N))

    x_p = jnp.pad(x, ((0, _NPAD - N), (0, 0)))
    h1s_p = _mm_scale(x_p, Wn1, dis_p[:, None], 1024)
    h1s = h1s_p[:N]
    s = _node_scatter_sc(h1s_p, row, col, Wn1.shape[1], E)[:N]
    q = _sc_scatter_vals(row, col, None, None, E, gtab=dis_p)
    wnode = dis * jnp.sum(q[:, 0], axis=0)[:N] + dis * dis
    nodesum = _node_reduce(s, h1s, dis[:, None], wnode[:, None],
                           bn1[None, :], 1000)

    u0, u1 = row, col
    nodes = edge_index.T.reshape(-1)
    order = jnp.argsort(nodes, stable=True).astype(jnp.int32)
    cnt = (cnt_row + cnt_col).astype(jnp.int32)
    first_pos = jnp.concatenate(
        [jnp.zeros((1,), jnp.int32), jnp.cumsum(cnt)[:-1].astype(jnp.int32)])
    counted_f = _sc_rank_counted(order, nodes, first_pos, cnt, 2 * E)
    counted = counted_f > 0.5
    c0, c1 = counted[0::2], counted[1::2]
    selfloop = u0 == u1
    w0 = (c0 | (selfloop & c1)).astype(jnp.float32)
    w1 = (c1 & (~selfloop)).astype(jnp.float32)
    both = c0 & c1
    eligible = both & (~selfloop)
    key = jnp.where(eligible,
                    jnp.minimum(u0, u1) * N + jnp.maximum(u0, u1),
                    N * N)
    p = jnp.argsort(key).astype(jnp.int32)
    key_s = key[p]
    dkey = key_s[1:] != key_s[:-1]
    idx_e = jnp.arange(E, dtype=jnp.int32)
    start_run = lax.cummax(
        jnp.where(jnp.concatenate([jnp.array([True]), dkey]), idx_e, 0))
    end_run = lax.cummin(
        jnp.where(jnp.concatenate([dkey, jnp.array([True])]), idx_e, E),
        reverse=True)
    fc2 = _sc_scatter_vals(u0, u1, w0, w1, E)
    Fc = jnp.sum(fc2[:, 0], axis=0)
    fcg = _sc_gather2(Fc, u0, u1, E)
    Fcu0, Fcu1 = fcg[0], fcg[1]
    Gc_s = (end_run - start_run + 1).astype(jnp.float32)
    Gc = jnp.zeros((E,), jnp.float32).at[p].set(Gc_s)
    degA = jnp.where(eligible, Fcu0 + Fcu1 - Gc - 1.0,
           jnp.where(both, Fcu0,
           jnp.where(c0, Fcu0 - 1.0,
           jnp.where(c1, Fcu1 - 1.0, 0.0))))
    dise = (degA + 1.0) ** -0.5

    We1_p = jnp.pad(We1, ((0, 0), (0, 128 - We1.shape[1])))
    f_pad = _mm_scale(edge_attr, We1_p, dise[:, None], 8000)
    f = f_pad[:, :We1.shape[1]]
    big = 4 * _NPAD
    t0 = jnp.where(w0 > 0, u0, big).astype(jnp.int32)
    t1 = jnp.where(w1 > 0, u1, big).astype(jnp.int32)
    F_p = _sc_f_scatter(f_pad, t0, t1, 128, E)
    fg = _sc_row_gather2(F_p, u0, u1, 128, E)
    Fu0 = fg[0, :, :We1.shape[1]]
    Fu1 = fg[1, :, :We1.shape[1]]
    f_s = f[p]
    cs2 = jnp.cumsum(f_s, axis=0)
    G_s = cs2[end_run] - (cs2 - f_s)[start_run]
    G = jnp.zeros((E, We1.shape[1]), jnp.float32).at[p].set(G_s)

    alpha0 = c0.astype(jnp.float32)
    alpha1 = (eligible | (c1 & ~c0)).astype(jnp.float32)
    alphaf = -((c0 | c1) & ~(both & selfloop)).astype(jnp.float32)
    cv2 = _sc_scatter_vals(u0, u1, dise * alpha0, dise * alpha1, E)
    C = jnp.sum(cv2[:, 0], axis=0)
    cg = _sc_gather2(C, u0, u1, E)
    tv_s = jnp.where(eligible, -dise, 0.0)[p]
    cs1 = jnp.cumsum(tv_s)
    T_s = cs1[end_run] - (cs1 - tv_s)[start_run]
    T = jnp.zeros((E,), jnp.float32).at[p].set(T_s)
    lam = dise * (w0 * cg[0] + w1 * cg[1] + T) + dise * dise * (alphaf + 1.0)

    elig_f = eligible.astype(jnp.float32)
    z = (dise * alpha0)[:, None] * Fu0 + (dise * alpha1)[:, None] * Fu1 \
        - (dise * elig_f)[:, None] * G + (dise * (alphaf + 1.0))[:, None] * f
    edgesum = _edge_reduce(z, lam[:, None], be1[None, :], 4000)

    return _tail(nodesum, edgesum, Wn2, We2, Wfc, bn2, be2, bfc, N, E)


# speedup vs baseline: 3.7918x; 1.0500x over previous
import functools

import jax
import jax.numpy as jnp
from jax import lax
from jax.experimental import pallas as pl
from jax.experimental.pallas import tpu as pltpu
from jax.experimental.pallas import tpu_sc as plsc

_NC = 2
_NS = 16
_L = 16


_NPAD = 10240
_HALF = 5120
_REAL_HALF = 5000
_DUMMY = 5100
_EK = 80


def _node_scatter_sc(h1s_p, row, col, d_feat, n_edges):
    epw = n_edges // _NS
    chunks = epw // _EK
    mesh = plsc.VectorSubcoreMesh(core_axis_name="c", subcore_axis_name="s")

    @functools.partial(
        pl.kernel,
        out_type=jax.ShapeDtypeStruct((_NPAD, d_feat), jnp.float32),
        mesh=mesh,
        scratch_types=dict(
            table=pltpu.VMEM_SHARED((_HALF, d_feat), jnp.float32),
            rowflat=pltpu.VMEM((epw,), jnp.int32),
            colflat=pltpu.VMEM((epw,), jnp.int32),
            lidx=pltpu.VMEM((chunks, _EK), jnp.int32),
            rbuf=pltpu.VMEM((_EK, d_feat), jnp.float32),
            zbuf=pltpu.VMEM((_EK, d_feat), jnp.float32),
            sem=pltpu.SemaphoreType.DMA,
        ),
    )
    def k(h1s_hbm, row_hbm, col_hbm, out_hbm, table, rowflat, colflat, lidx,
          rbuf, zbuf, sem):
        cc = lax.axis_index("c")
        tid = lax.axis_index("s")
        base = cc * _REAL_HALF
        nb = d_feat // _L

        def zrow(r, _):
            for j in range(nb):
                zbuf[r, pl.ds(j * _L, _L)] = jnp.zeros((_L,), jnp.float32)
            return 0
        lax.fori_loop(0, _EK, zrow, 0)
        rows_per_tile = _HALF // _NS
        for j in range(rows_per_tile // _EK):
            pltpu.sync_copy(zbuf, table.at[pl.ds(tid * rows_per_tile + j * _EK,
                                                 _EK)])
        plsc.subcore_barrier()

        e0 = tid * epw
        pltpu.sync_copy(row_hbm.at[pl.ds(e0, epw)], rowflat)
        pltpu.sync_copy(col_hbm.at[pl.ds(e0, epw)], colflat)

        def lidx_body(g, _):
            for j in range(_EK // _L):
                cv = colflat[pl.ds(g * _EK + j * _L, _L)]
                lc = cv - base
                m = (lc >= 0) & (lc < _REAL_HALF)
                lidx[g, pl.ds(j * _L, _L)] = jnp.where(m, lc, _DUMMY)
            return 0
        lax.fori_loop(0, chunks, lidx_body, 0)

        def chunk_body(g, _):
            pltpu.async_copy(h1s_hbm.at[rowflat.at[pl.ds(g * _EK, _EK)]],
                             rbuf, sem).wait()
            pltpu.sync_copy(rbuf, table.at[lidx.at[g]], add=True)
            return 0
        lax.fori_loop(0, chunks, chunk_body, 0)
        plsc.subcore_barrier()

        for j in range(rows_per_tile // _EK):
            local = tid * rows_per_tile + j * _EK
            @pl.when(local < _REAL_HALF)
            def _():
                pltpu.sync_copy(table.at[pl.ds(local, _EK)], rbuf)
                pltpu.sync_copy(rbuf, out_hbm.at[pl.ds(base + local, _EK)])

    return k(h1s_p, row, col)


def _mesh():
    return plsc.VectorSubcoreMesh(core_axis_name="c", subcore_axis_name="s")


def _zero1d(ref, n):
    def b(i, _):
        ref[pl.ds(i * _L, _L)] = jnp.zeros((_L,), jnp.float32)
        return 0
    lax.fori_loop(0, n // _L, b, 0)


def _emit_tables(widx, tabs, out_hbm):
    for j, t in enumerate(tabs):
        pltpu.sync_copy(t, out_hbm.at[widx, j])


def _sc_scatter_counts(idxa, idxb, e_cnt):
    ept = e_cnt // (_NC * _NS)
    full, rem = ept // _L, ept % _L
    seg = _NPAD // _NS

    @functools.partial(
        pl.kernel,
        out_type=jax.ShapeDtypeStruct((_NC * _NS, 2, _NPAD), jnp.float32),
        mesh=_mesh(),
        compiler_params=pltpu.CompilerParams(needs_layout_passes=False),
        scratch_types=dict(
            tab0=pltpu.VMEM((_NPAD,), jnp.float32),
            tab1=pltpu.VMEM((_NPAD,), jnp.float32),
            ia=pltpu.VMEM((ept + _L,), jnp.int32),
            ib=pltpu.VMEM((ept + _L,), jnp.int32),
        ),
    )
    def k(ia_hbm, ib_hbm, out_hbm, tab0, tab1, ia, ib):
        cc = lax.axis_index("c")
        tid = lax.axis_index("s")
        e0 = (cc * _NS + tid) * ept
        _zero1d(tab0, _NPAD)
        _zero1d(tab1, _NPAD)
        pltpu.sync_copy(ia_hbm.at[pl.ds(e0, ept)], ia.at[pl.ds(0, ept)])
        pltpu.sync_copy(ib_hbm.at[pl.ds(e0, ept)], ib.at[pl.ds(0, ept)])
        ones = jnp.ones((_L,), jnp.float32)

        def b(g, _):
            plsc.addupdate_scatter(tab0, [ia[pl.ds(g * _L, _L)]], ones)
            plsc.addupdate_scatter(tab1, [ib[pl.ds(g * _L, _L)]], ones)
            return 0
        lax.fori_loop(0, full, b, 0)
        if rem:
            m = lax.iota(jnp.int32, _L) < rem
            plsc.addupdate_scatter(tab0, [ia[pl.ds(full * _L, _L)]], ones, mask=m)
            plsc.addupdate_scatter(tab1, [ib[pl.ds(full * _L, _L)]], ones, mask=m)
        _emit_tables(cc * _NS + tid, [tab0, tab1], out_hbm)

    return k(idxa, idxb)


def _sc_scatter_vals(idxa, idxb, vala, valb, e_cnt, gtab=None):
    ept = e_cnt // (_NC * _NS)
    full, rem = ept // _L, ept % _L
    seg = _NPAD // _NS
    gmode = gtab is not None
    scr = dict(
        tab0=pltpu.VMEM((_NPAD,), jnp.float32),
        ia=pltpu.VMEM((ept + _L,), jnp.int32),
        ib=pltpu.VMEM((ept + _L,), jnp.int32),
    )
    if gmode:
        scr["gbuf"] = pltpu.VMEM((_NPAD,), jnp.float32)
    else:
        scr["va"] = pltpu.VMEM((ept + _L,), jnp.float32)
        scr["vb"] = pltpu.VMEM((ept + _L,), jnp.float32)

    def body(cc, tid, ia_hbm, ib_hbm, va_hbm, vb_hbm, g_hbm, out_hbm,
             tab0, ia, ib, gbuf=None, va=None, vb=None):
        e0 = (cc * _NS + tid) * ept
        _zero1d(tab0, _NPAD)
        pltpu.sync_copy(ia_hbm.at[pl.ds(e0, ept)], ia.at[pl.ds(0, ept)])
        pltpu.sync_copy(ib_hbm.at[pl.ds(e0, ept)], ib.at[pl.ds(0, ept)])
        if gmode:
            pltpu.sync_copy(g_hbm, gbuf)
        else:
            pltpu.sync_copy(va_hbm.at[pl.ds(e0, ept)], va.at[pl.ds(0, ept)])
            pltpu.sync_copy(vb_hbm.at[pl.ds(e0, ept)], vb.at[pl.ds(0, ept)])
        allm = lax.iota(jnp.int32, _L) < _L

        def step(g, m):
            av = ia[pl.ds(g * _L, _L)]
            bv = ib[pl.ds(g * _L, _L)]
            if gmode:
                dv = plsc.load_gather(gbuf, [bv], mask=m)
                plsc.addupdate_scatter(tab0, [av], dv, mask=m)
            else:
                plsc.addupdate_scatter(tab0, [av], va[pl.ds(g * _L, _L)], mask=m)
                plsc.addupdate_scatter(tab0, [bv], vb[pl.ds(g * _L, _L)], mask=m)

        def b(g, _):
            step(g, allm)
            return 0
        lax.fori_loop(0, full, b, 0)
        if rem:
            step(full, lax.iota(jnp.int32, _L) < rem)
        _emit_tables(cc * _NS + tid, [tab0], out_hbm)

    out_type = jax.ShapeDtypeStruct((_NC * _NS, 1, _NPAD), jnp.float32)
    cp = pltpu.CompilerParams(needs_layout_passes=False)
    if gmode:
        @functools.partial(pl.kernel, out_type=out_type, mesh=_mesh(),
                           scratch_types=scr, compiler_params=cp)
        def k(ia_hbm, ib_hbm, g_hbm, out_hbm, tab0, ia, ib, gbuf):
            cc = lax.axis_index("c")
            tid = lax.axis_index("s")
            body(cc, tid, ia_hbm, ib_hbm, None, None, g_hbm, out_hbm,
                 tab0, ia, ib, gbuf=gbuf)
        return k(idxa, idxb, gtab)
    else:
        @functools.partial(pl.kernel, out_type=out_type, mesh=_mesh(),
                           scratch_types=scr, compiler_params=cp)
        def k(ia_hbm, ib_hbm, va_hbm, vb_hbm, out_hbm, tab0, ia, ib, va, vb):
            cc = lax.axis_index("c")
            tid = lax.axis_index("s")
            body(cc, tid, ia_hbm, ib_hbm, va_hbm, vb_hbm, None, out_hbm,
                 tab0, ia, ib, va=va, vb=vb)
        return k(idxa, idxb, vala, valb)


def _sc_gather2(tab, idxa, idxb, e_cnt):
    ept = e_cnt // (_NC * _NS)
    full, rem = ept // _L, ept % _L

    @functools.partial(
        pl.kernel,
        out_type=jax.ShapeDtypeStruct((2 * e_cnt,), jnp.float32),
        mesh=_mesh(),
        compiler_params=pltpu.CompilerParams(needs_layout_passes=False),
        scratch_types=dict(
            tabbuf=pltpu.VMEM((_NPAD,), jnp.float32),
            ia=pltpu.VMEM((ept + _L,), jnp.int32),
            ib=pltpu.VMEM((ept + _L,), jnp.int32),
            oa=pltpu.VMEM((ept + _L,), jnp.float32),
            ob=pltpu.VMEM((ept + _L,), jnp.float32),
        ),
    )
    def k(tab_hbm, ia_hbm, ib_hbm, out_hbm, tabbuf, ia, ib, oa, ob):
        cc = lax.axis_index("c")
        tid = lax.axis_index("s")
        e0 = (cc * _NS + tid) * ept
        pltpu.sync_copy(tab_hbm, tabbuf)
        pltpu.sync_copy(ia_hbm.at[pl.ds(e0, ept)], ia.at[pl.ds(0, ept)])
        pltpu.sync_copy(ib_hbm.at[pl.ds(e0, ept)], ib.at[pl.ds(0, ept)])
        nch = full + (1 if rem else 0)

        def b(g, _):
            m = lax.iota(jnp.int32, _L) + g * _L < ept
            oa[pl.ds(g * _L, _L)] = plsc.load_gather(
                tabbuf, [ia[pl.ds(g * _L, _L)]], mask=m)
            ob[pl.ds(g * _L, _L)] = plsc.load_gather(
                tabbuf, [ib[pl.ds(g * _L, _L)]], mask=m)
            return 0
        lax.fori_loop(0, nch, b, 0)
        pltpu.sync_copy(oa.at[pl.ds(0, ept)], out_hbm.at[pl.ds(e0, ept)])
        pltpu.sync_copy(ob.at[pl.ds(0, ept)],
                        out_hbm.at[pl.ds(e_cnt + e0, ept)])

    return k(tab, idxa, idxb).reshape(2, e_cnt)


def _sc_rank_counted(order, nodes, first_pos, cnt, n_slots):
    spw = n_slots // (_NC * _NS)
    ck = _EK
    chunks = spw // ck

    @functools.partial(
        pl.kernel,
        out_type=jax.ShapeDtypeStruct((n_slots,), jnp.float32),
        mesh=_mesh(),
        compiler_params=pltpu.CompilerParams(needs_layout_passes=False),
        scratch_types=dict(
            fptab=pltpu.VMEM((_NPAD,), jnp.int32),
            cnttab=pltpu.VMEM((_NPAD,), jnp.int32),
            oflat=pltpu.VMEM((spw,), jnp.int32),
            o2d=pltpu.VMEM((chunks, ck), jnp.int32),
            nbuf=pltpu.VMEM((ck,), jnp.int32),
            cbuf=pltpu.VMEM((ck,), jnp.float32),
            sem=pltpu.SemaphoreType.DMA,
        ),
    )
    def k(order_hbm, nodes_hbm, fp_hbm, cnt_hbm, out_hbm, fptab, cnttab,
          oflat, o2d, nbuf, cbuf, sem):
        cc = lax.axis_index("c")
        tid = lax.axis_index("s")
        s0 = (cc * _NS + tid) * spw
        pltpu.sync_copy(fp_hbm, fptab)
        pltpu.sync_copy(cnt_hbm, cnttab)
        pltpu.sync_copy(order_hbm.at[pl.ds(s0, spw)], oflat)

        def stage(g, _):
            pltpu.sync_copy(order_hbm.at[pl.ds(s0 + g * ck, ck)], o2d.at[g])
            return 0
        lax.fori_loop(0, chunks, stage, 0)

        def b(g, _):
            pltpu.async_copy(nodes_hbm.at[oflat.at[pl.ds(g * ck, ck)]],
                             nbuf, sem).wait()
            for j in range(ck // _L):
                nv = nbuf[pl.ds(j * _L, _L)]
                fp = plsc.load_gather(fptab, [nv])
                cv = plsc.load_gather(cnttab, [nv])
                ivec = (s0 + g * ck + j * _L) + lax.iota(jnp.int32, _L)
                ok = (ivec - fp) < jnp.minimum(cv, 100)
                cbuf[pl.ds(j * _L, _L)] = jnp.where(ok, 1.0, 0.0)
            pltpu.sync_copy(cbuf, out_hbm.at[o2d.at[g]])
            return 0
        lax.fori_loop(0, chunks, b, 0)

    return k(order, nodes, first_pos, cnt)


def _sc_f_scatter(rows, t0, t1, d_feat, n_edges):
    epw = n_edges // _NS
    chunks = epw // _EK

    @functools.partial(
        pl.kernel,
        out_type=jax.ShapeDtypeStruct((_NPAD, d_feat), jnp.float32),
        mesh=_mesh(),
        scratch_types=dict(
            table=pltpu.VMEM_SHARED((_HALF, d_feat), jnp.float32),
            t0flat=pltpu.VMEM((epw,), jnp.int32),
            t1flat=pltpu.VMEM((epw,), jnp.int32),
            l0=pltpu.VMEM((chunks, _EK), jnp.int32),
            l1=pltpu.VMEM((chunks, _EK), jnp.int32),
            rbuf=pltpu.VMEM((_EK, d_feat), jnp.float32),
            zbuf=pltpu.VMEM((_EK, d_feat), jnp.float32),
        ),
    )
    def k(rows_hbm, t0_hbm, t1_hbm, out_hbm, table, t0flat, t1flat, l0, l1,
          rbuf, zbuf):
        cc = lax.axis_index("c")
        tid = lax.axis_index("s")
        base = cc * _REAL_HALF
        nb = d_feat // _L

        def zrow(r, _):
            for j in range(nb):
                zbuf[r, pl.ds(j * _L, _L)] = jnp.zeros((_L,), jnp.float32)
            return 0
        lax.fori_loop(0, _EK, zrow, 0)
        rpt = _HALF // _NS
        for j in range(rpt // _EK):
            pltpu.sync_copy(zbuf, table.at[pl.ds(tid * rpt + j * _EK, _EK)])
        plsc.subcore_barrier()

        e0 = tid * epw
        pltpu.sync_copy(t0_hbm.at[pl.ds(e0, epw)], t0flat)
        pltpu.sync_copy(t1_hbm.at[pl.ds(e0, epw)], t1flat)

        def lb(g, _):
            for j in range(_EK // _L):
                for (src, dst) in ((t0flat, l0), (t1flat, l1)):
                    tv = src[pl.ds(g * _EK + j * _L, _L)]
                    lc = tv - base
                    m = (lc >= 0) & (lc < _REAL_HALF)
                    dst[g, pl.ds(j * _L, _L)] = jnp.where(m, lc, _DUMMY)
            return 0
        lax.fori_loop(0, chunks, lb, 0)

        def b(g, _):
            pltpu.sync_copy(rows_hbm.at[pl.ds(e0 + g * _EK, _EK)], rbuf)
            pltpu.sync_copy(rbuf, table.at[l0.at[g]], add=True)
            pltpu.sync_copy(rbuf, table.at[l1.at[g]], add=True)
            return 0
        lax.fori_loop(0, chunks, b, 0)
        plsc.subcore_barrier()

        for j in range(rpt // _EK):
            local = tid * rpt + j * _EK
            @pl.when(local < _REAL_HALF)
            def _():
                pltpu.sync_copy(table.at[pl.ds(local, _EK)], rbuf)
                pltpu.sync_copy(rbuf, out_hbm.at[pl.ds(base + local, _EK)])

    return k(rows, t0, t1)


def _sc_row_gather2(tab_p, idxa, idxb, d_feat, e_cnt):
    ept = e_cnt // (_NC * _NS)
    ck = 40
    chunks = ept // ck

    @functools.partial(
        pl.kernel,
        out_type=jax.ShapeDtypeStruct((2, e_cnt, d_feat), jnp.float32),
        mesh=_mesh(),
        scratch_types=dict(
            ia=pltpu.VMEM((ept,), jnp.int32),
            ib=pltpu.VMEM((ept,), jnp.int32),
            buf=pltpu.VMEM((ck, d_feat), jnp.float32),
            sem=pltpu.SemaphoreType.DMA,
        ),
    )
    def k(tab_hbm, ia_hbm, ib_hbm, out_hbm, ia, ib, buf, sem):
        cc = lax.axis_index("c")
        tid = lax.axis_index("s")
        e0 = (cc * _NS + tid) * ept
        pltpu.sync_copy(ia_hbm.at[pl.ds(e0, ept)], ia)
        pltpu.sync_copy(ib_hbm.at[pl.ds(e0, ept)], ib)

        def b(g, _):
            pltpu.async_copy(tab_hbm.at[ia.at[pl.ds(g * ck, ck)]],
                             buf, sem).wait()
            pltpu.sync_copy(buf, out_hbm.at[0, pl.ds(e0 + g * ck, ck)])
            pltpu.async_copy(tab_hbm.at[ib.at[pl.ds(g * ck, ck)]],
                             buf, sem).wait()
            pltpu.sync_copy(buf, out_hbm.at[1, pl.ds(e0 + g * ck, ck)])
            return 0
        lax.fori_loop(0, chunks, b, 0)

    return k(tab_p, idxa, idxb)


def _mm_scale_body(x_ref, w_ref, scale_ref, out_ref):
    out_ref[...] = scale_ref[...] * jnp.dot(
        x_ref[...], w_ref[...], preferred_element_type=jnp.float32)


def _mm_scale(x, w, scale, block_rows):
    n, k = x.shape
    m = w.shape[1]
    grid = n // block_rows
    return pl.pallas_call(
        _mm_scale_body,
        grid=(grid,),
        in_specs=[
            pl.BlockSpec((block_rows, k), lambda i: (i, 0)),
            pl.BlockSpec((k, m), lambda i: (0, 0)),
            pl.BlockSpec((block_rows, 1), lambda i: (i, 0)),
        ],
        out_specs=pl.BlockSpec((block_rows, m), lambda i: (i, 0)),
        out_shape=jax.ShapeDtypeStruct((n, m), jnp.float32),
    )(x, w, scale)


def _node_reduce_body(s_ref, h1s_ref, dis_ref, wn_ref, b_ref, out_ref):
    @pl.when(pl.program_id(0) == 0)
    def _():
        out_ref[...] = jnp.zeros_like(out_ref)

    h = jax.nn.sigmoid(dis_ref[...] * (s_ref[...] + h1s_ref[...]) + b_ref[...])
    out_ref[...] += jnp.dot(wn_ref[...].T, h, preferred_element_type=jnp.float32)


def _node_reduce(s, h1s, dis, wn, b, block_rows):
    n, d = s.shape
    grid = n // block_rows
    return pl.pallas_call(
        _node_reduce_body,
        grid=(grid,),
        in_specs=[
            pl.BlockSpec((block_rows, d), lambda i: (i, 0)),
            pl.BlockSpec((block_rows, d), lambda i: (i, 0)),
            pl.BlockSpec((block_rows, 1), lambda i: (i, 0)),
            pl.BlockSpec((block_rows, 1), lambda i: (i, 0)),
            pl.BlockSpec((1, d), lambda i: (0, 0)),
        ],
        out_specs=pl.BlockSpec((1, d), lambda i: (0, 0)),
        out_shape=jax.ShapeDtypeStruct((1, d), jnp.float32),
    )(s, h1s, dis, wn, b)


def _edge_reduce_body(z_ref, lam_ref, b_ref, out_ref):
    @pl.when(pl.program_id(0) == 0)
    def _():
        out_ref[...] = jnp.zeros_like(out_ref)

    e = jax.nn.sigmoid(z_ref[...] + b_ref[...])
    out_ref[...] += jnp.dot(lam_ref[...].T, e, preferred_element_type=jnp.float32)


def _edge_reduce(z, lam, b, block_rows):
    n, d = z.shape
    grid = n // block_rows
    return pl.pallas_call(
        _edge_reduce_body,
        grid=(grid,),
        in_specs=[pl.BlockSpec((block_rows, d), lambda i: (i, 0)),
                  pl.BlockSpec((block_rows, 1), lambda i: (i, 0)),
                  pl.BlockSpec((1, d), lambda i: (0, 0))],
        out_specs=pl.BlockSpec((1, d), lambda i: (0, 0)),
        out_shape=jax.ShapeDtypeStruct((1, d), jnp.float32),
    )(z, lam, b)


def _tail_body(ns_ref, es_ref, wn2_ref, we2_ref, wfc_ref, bn2_ref, be2_ref,
               bfc_ref, nn_ref, ee_ref, out_ref):
    node_rep = jnp.dot(ns_ref[...], wn2_ref[...],
                       preferred_element_type=jnp.float32) / nn_ref[0, 0] + bn2_ref[...]
    edge_rep = jnp.dot(es_ref[...], we2_ref[...],
                       preferred_element_type=jnp.float32) / ee_ref[0, 0] + be2_ref[...]
    rep = jnp.concatenate([node_rep, edge_rep], axis=-1)
    out_ref[...] = jnp.dot(rep, wfc_ref[...],
                           preferred_element_type=jnp.float32) + bfc_ref[...]


def _tail(nodesum, edgesum, Wn2, We2, Wfc, bn2, be2, bfc, n_nodes, n_edges):
    nn = jnp.full((1, 1), float(n_nodes), jnp.float32)
    ee = jnp.full((1, 1), float(n_edges), jnp.float32)
    full = lambda shape: pl.BlockSpec(shape, lambda: (0,) * len(shape))
    return pl.pallas_call(
        _tail_body,
        in_specs=[full(nodesum.shape), full(edgesum.shape), full(Wn2.shape),
                  full(We2.shape), full(Wfc.shape), full((1, Wn2.shape[1])),
                  full((1, We2.shape[1])), full((1, Wfc.shape[1])),
                  full((1, 1)), full((1, 1))],
        out_specs=full((1, Wfc.shape[1])),
        out_shape=jax.ShapeDtypeStruct((1, Wfc.shape[1]), jnp.float32),
    )(nodesum, edgesum, Wn2, We2, Wfc, bn2[None, :], be2[None, :], bfc[None, :],
      nn, ee)


def kernel(x, edge_index, edge_attr, batch, Wn1, bn1, Wn2, bn2, We1, be1,
           We2, be2, Wfc, bfc):
    N = x.shape[0]
    E = edge_index.shape[1]
    row, col = edge_index[0], edge_index[1]

    ab = _sc_scatter_counts(row, col, E)
    cnt_row = jnp.sum(ab[:, 0], axis=0)
    cnt_col = jnp.sum(ab[:, 1], axis=0)
    deg = cnt_col[:N] + 1.0
    dis = deg ** -0.5
    dis_p = jnp.pad(dis, (0, _NPAD - N))

    x_p = jnp.pad(x, ((0, _NPAD - N), (0, 0)))
    h1s_p = _mm_scale(x_p, Wn1, dis_p[:, None], 1024)
    h1s = h1s_p[:N]
    s = _node_scatter_sc(h1s_p, row, col, Wn1.shape[1], E)[:N]
    q = _sc_scatter_vals(row, col, None, None, E, gtab=dis_p)
    wnode = dis * jnp.sum(q[:, 0], axis=0)[:N] + dis * dis
    nodesum = _node_reduce(s, h1s, dis[:, None], wnode[:, None],
                           bn1[None, :], 1000)

    u0, u1 = row, col
    nodes = edge_index.T.reshape(-1)
    order = jnp.argsort(nodes, stable=True).astype(jnp.int32)
    cnt = (cnt_row + cnt_col).astype(jnp.int32)
    first_pos = jnp.concatenate(
        [jnp.zeros((1,), jnp.int32), jnp.cumsum(cnt)[:-1].astype(jnp.int32)])
    counted_f = _sc_rank_counted(order, nodes, first_pos, cnt, 2 * E)
    counted = counted_f > 0.5
    c0, c1 = counted[0::2], counted[1::2]
    selfloop = u0 == u1
    w0 = (c0 | (selfloop & c1)).astype(jnp.float32)
    w1 = (c1 & (~selfloop)).astype(jnp.float32)
    both = c0 & c1
    eligible = both & (~selfloop)
    key = jnp.where(eligible,
                    jnp.minimum(u0, u1) * N + jnp.maximum(u0, u1),
                    N * N)
    p = jnp.argsort(key).astype(jnp.int32)
    key_s = key[p]
    dkey = key_s[1:] != key_s[:-1]
    idx_e = jnp.arange(E, dtype=jnp.int32)
    start_run = lax.cummax(
        jnp.where(jnp.concatenate([jnp.array([True]), dkey]), idx_e, 0))
    end_run = lax.cummin(
        jnp.where(jnp.concatenate([dkey, jnp.array([True])]), idx_e, E),
        reverse=True)
    fc2 = _sc_scatter_vals(u0, u1, w0, w1, E)
    Fc = jnp.sum(fc2[:, 0], axis=0)
    fcg = _sc_gather2(Fc, u0, u1, E)
    Fcu0, Fcu1 = fcg[0], fcg[1]
    Gc_s = (end_run - start_run + 1).astype(jnp.float32)
    Gc = jnp.zeros((E,), jnp.float32).at[p].set(Gc_s)
    degA = jnp.where(eligible, Fcu0 + Fcu1 - Gc - 1.0,
           jnp.where(both, Fcu0,
           jnp.where(c0, Fcu0 - 1.0,
           jnp.where(c1, Fcu1 - 1.0, 0.0))))
    dise = (degA + 1.0) ** -0.5

    We1_p = jnp.pad(We1, ((0, 0), (0, 128 - We1.shape[1])))
    f_pad = _mm_scale(edge_attr, We1_p, dise[:, None], 8000)
    f = f_pad[:, :We1.shape[1]]
    big = 4 * _NPAD
    t0 = jnp.where(w0 > 0, u0, big).astype(jnp.int32)
    t1 = jnp.where(w1 > 0, u1, big).astype(jnp.int32)
    F_p = _sc_f_scatter(f_pad, t0, t1, 128, E)
    fg = _sc_row_gather2(F_p, u0, u1, 128, E)
    Fu0 = fg[0, :, :We1.shape[1]]
    Fu1 = fg[1, :, :We1.shape[1]]
    tv_s = jnp.where(eligible, -dise, 0.0)[p]
    f_s = jnp.concatenate([f[p], tv_s[:, None]], axis=1)
    cs2 = jnp.cumsum(f_s, axis=0)
    GT_s = cs2[end_run] - (cs2 - f_s)[start_run]
    GT = jnp.zeros((E, We1.shape[1] + 1), jnp.float32).at[p].set(GT_s)
    G = GT[:, :We1.shape[1]]

    alpha0 = c0.astype(jnp.float32)
    alpha1 = (eligible | (c1 & ~c0)).astype(jnp.float32)
    alphaf = -((c0 | c1) & ~(both & selfloop)).astype(jnp.float32)
    cv2 = _sc_scatter_vals(u0, u1, dise * alpha0, dise * alpha1, E)
    C = jnp.sum(cv2[:, 0], axis=0)
    cg = _sc_gather2(C, u0, u1, E)
    T = GT[:, We1.shape[1]]
    lam = dise * (w0 * cg[0] + w1 * cg[1] + T) + dise * dise * (alphaf + 1.0)

    elig_f = eligible.astype(jnp.float32)
    z = (dise * alpha0)[:, None] * Fu0 + (dise * alpha1)[:, None] * Fu1 \
        - (dise * elig_f)[:, None] * G + (dise * (alphaf + 1.0))[:, None] * f
    edgesum = _edge_reduce(z, lam[:, None], be1[None, :], 4000)

    return _tail(nodesum, edgesum, Wn2, We2, Wfc, bn2, be2, bfc, N, E)
